# baseline (speedup 1.0000x reference)
import functools

import jax
import jax.numpy as jnp
from jax import lax
from jax.experimental import pallas as pl
from jax.experimental.pallas import tpu as pltpu
from jax.experimental.pallas import tpu_sc as plsc

N_NODES = 10000
NUM_EDGES = 320000
NUM_GRAPHS = 64
NCORE = 2
NSUB = 16
NW = NCORE * NSUB
CHUNK = 128
K_CHUNKS = 80
NBUF = 4
NGRP = K_CHUNKS // NBUF
CAP = NW * K_CHUNKS * CHUNK
N_PAD = 10112
ROWS_PER_TILE = N_PAD // NSUB
JUNK_ROW = 10100
BLK = 1000


def _mesh():
    return plsc.VectorSubcoreMesh(core_axis_name="c", subcore_axis_name="s")


def _deg_call(dst_p, ones_rows, zeros16):
    @functools.partial(
        pl.kernel,
        mesh=_mesh(),
        out_type=jax.ShapeDtypeStruct((NCORE, N_PAD, 16), jnp.float32),
        scratch_types=[
            pltpu.VMEM((K_CHUNKS, CHUNK), jnp.int32),
            pltpu.VMEM((CHUNK, 16), jnp.float32),
            pltpu.VMEM_SHARED((N_PAD, 16), jnp.float32),
            pltpu.SemaphoreType.DMA,
        ],
        compiler_params=pltpu.CompilerParams(use_tc_tiling_on_sc=False),
    )
    def deg_k(dst_hbm, ones_hbm, zeros_hbm, out_hbm, idx_v, ones_v, bins_sh,
              sem):
        c = lax.axis_index("c")
        s = lax.axis_index("s")
        wid = c * NSUB + s
        r0 = pl.multiple_of(s * ROWS_PER_TILE, 8)
        pltpu.sync_copy(zeros_hbm.at[pl.ds(r0, ROWS_PER_TILE)],
                        bins_sh.at[pl.ds(r0, ROWS_PER_TILE)])
        pltpu.sync_copy(ones_hbm, ones_v)
        pltpu.sync_copy(dst_hbm.at[wid], idx_v)
        plsc.subcore_barrier()

        def body(t, carry):
            for b in range(8):
                pltpu.async_copy(ones_v, bins_sh.at[idx_v.at[t * 8 + b]],
                                 sem, add=True)

            @pl.when(t > 0)
            def _():
                for b in range(8):
                    pltpu.make_async_copy(
                        ones_v, bins_sh.at[idx_v.at[b]], sem).wait()

            return carry

        lax.fori_loop(0, K_CHUNKS // 8, body, 0)
        for b in range(8):
            pltpu.make_async_copy(ones_v, bins_sh.at[idx_v.at[b]], sem).wait()
        plsc.subcore_barrier()
        pltpu.sync_copy(bins_sh.at[pl.ds(r0, ROWS_PER_TILE)],
                        out_hbm.at[c, pl.ds(r0, ROWS_PER_TILE)])

    return deg_k(dst_p, ones_rows, zeros16)


def _prop_call(u, src_p, dst_p, zeros, d):
    @functools.partial(
        pl.kernel,
        mesh=_mesh(),
        out_type=jax.ShapeDtypeStruct((NCORE, N_PAD, d), jnp.float32),
        scratch_types=[
            pltpu.VMEM((K_CHUNKS, CHUNK), jnp.int32),
            pltpu.VMEM((K_CHUNKS, CHUNK), jnp.int32),
            pltpu.VMEM((2 * NBUF, CHUNK, d), jnp.float32),
            pltpu.VMEM_SHARED((N_PAD, d), jnp.float32),
        ] + [pltpu.SemaphoreType.DMA] * (4 * NBUF),
        compiler_params=pltpu.CompilerParams(use_tc_tiling_on_sc=False),
    )
    def prop_k(u_hbm, src_hbm, dst_hbm, zeros_hbm, out_hbm,
               src_v, dst_v, rows_v, acc_sh, *sems):
        gsem = sems[:2 * NBUF]
        ssem = sems[2 * NBUF:]
        c = lax.axis_index("c")
        s = lax.axis_index("s")
        wid = c * NSUB + s
        r0 = pl.multiple_of(s * ROWS_PER_TILE, 8)
        pltpu.sync_copy(zeros_hbm.at[pl.ds(r0, ROWS_PER_TILE)],
                        acc_sh.at[pl.ds(r0, ROWS_PER_TILE)])
        pltpu.sync_copy(src_hbm.at[wid], src_v)
        pltpu.sync_copy(dst_hbm.at[wid], dst_v)
        plsc.subcore_barrier()

        def fire_gather(slot, j):
            pltpu.async_copy(u_hbm.at[src_v.at[j]], rows_v.at[slot],
                             gsem[slot])

        def wait_gather(slot, j):
            pltpu.make_async_copy(u_hbm.at[src_v.at[j]], rows_v.at[slot],
                                  gsem[slot]).wait()

        for b in range(2 * NBUF):
            fire_gather(b, b)

        def body(t, carry):
            j0 = 2 * NBUF * t
            for half in range(2):
                off = half * NBUF
                for b in range(NBUF):
                    wait_gather(off + b, j0 + off + b)
                scps = [
                    pltpu.async_copy(rows_v.at[off + b],
                                     acc_sh.at[dst_v.at[j0 + off + b]],
                                     ssem[off + b], add=True)
                    for b in range(NBUF)
                ]
                for d_ in scps:
                    d_.wait()

                @pl.when(t < NGRP // 2 - 1)
                def _():
                    for b in range(NBUF):
                        fire_gather(off + b, j0 + 2 * NBUF + off + b)

            return carry

        lax.fori_loop(0, NGRP // 2, body, 0)
        plsc.subcore_barrier()
        pltpu.sync_copy(acc_sh.at[pl.ds(r0, ROWS_PER_TILE)],
                        out_hbm.at[c, pl.ds(r0, ROWS_PER_TILE)])

    return prop_k(u, src_p, dst_p, zeros)


def _tc1_call(bins, x, W1):
    def body(bins_ref, x_ref, w_ref, u_ref, dis_ref):
        deg = bins_ref[0][:, 0:1] + bins_ref[1][:, 0:1] + 1.0
        dis = lax.rsqrt(deg)
        h = jnp.dot(x_ref[...], w_ref[...], preferred_element_type=jnp.float32)
        u_ref[...] = h * dis
        dis_ref[...] = dis

    return pl.pallas_call(
        body,
        grid=(N_NODES // BLK,),
        in_specs=[
            pl.BlockSpec((NCORE, BLK, 16), lambda i: (0, i, 0)),
            pl.BlockSpec((BLK, 128), lambda i: (i, 0)),
            pl.BlockSpec((128, 64), lambda i: (0, 0)),
        ],
        out_specs=[
            pl.BlockSpec((BLK, 64), lambda i: (i, 0)),
            pl.BlockSpec((BLK, 1), lambda i: (i, 0)),
        ],
        out_shape=[
            jax.ShapeDtypeStruct((N_NODES, 64), jnp.float32),
            jax.ShapeDtypeStruct((N_NODES, 1), jnp.float32),
        ],
    )(bins, x, W1)


def _tc2_call(p, u1, dis, b1, W2):
    def body(p_ref, u1_ref, dis_ref, b1_ref, w2_ref, u2_ref):
        sfull = p_ref[0] + p_ref[1] + u1_ref[...]
        o = jnp.maximum(sfull * dis_ref[...] + b1_ref[...], 0.0)
        u2_ref[...] = jnp.dot(o, w2_ref[...],
                              preferred_element_type=jnp.float32) * dis_ref[...]

    return pl.pallas_call(
        body,
        grid=(N_NODES // BLK,),
        in_specs=[
            pl.BlockSpec((NCORE, BLK, 64), lambda i: (0, i, 0)),
            pl.BlockSpec((BLK, 64), lambda i: (i, 0)),
            pl.BlockSpec((BLK, 1), lambda i: (i, 0)),
            pl.BlockSpec((1, 64), lambda i: (0, 0)),
            pl.BlockSpec((64, 32), lambda i: (0, 0)),
        ],
        out_specs=pl.BlockSpec((BLK, 32), lambda i: (i, 0)),
        out_shape=jax.ShapeDtypeStruct((N_NODES, 32), jnp.float32),
    )(p, u1, dis, b1, W2)


def _tc3_call(p, u2, dis, b2, batch_r):
    def body(p_ref, u2_ref, dis_ref, b2_ref, bt_ref, out_ref):
        h2 = (p_ref[0] + p_ref[1] + u2_ref[...]) * dis_ref[...] + b2_ref[...]
        bt = bt_ref[0]
        oh = (lax.broadcasted_iota(jnp.int32, (NUM_GRAPHS, BLK), 0)
              == bt).astype(jnp.float32)
        acc = jnp.dot(oh, h2, preferred_element_type=jnp.float32)
        i = pl.program_id(0)

        @pl.when(i == 0)
        def _():
            out_ref[...] = acc

        @pl.when(i != 0)
        def _():
            out_ref[...] += acc

    return pl.pallas_call(
        body,
        grid=(N_NODES // BLK,),
        in_specs=[
            pl.BlockSpec((NCORE, BLK, 32), lambda i: (0, i, 0)),
            pl.BlockSpec((BLK, 32), lambda i: (i, 0)),
            pl.BlockSpec((BLK, 1), lambda i: (i, 0)),
            pl.BlockSpec((1, 32), lambda i: (0, 0)),
            pl.BlockSpec((1, 1, BLK), lambda i: (i, 0, 0)),
        ],
        out_specs=pl.BlockSpec((NUM_GRAPHS, 32), lambda i: (0, 0)),
        out_shape=jax.ShapeDtypeStruct((NUM_GRAPHS, 32), jnp.float32),
    )(p, u2, dis, b2, batch_r)


def kernel(x, edge_index, batch, W1, b1, W2, b2):
    src = edge_index[0].astype(jnp.int32)
    dst = edge_index[1].astype(jnp.int32)
    pad = CAP - NUM_EDGES
    src_p = jnp.concatenate(
        [src, jnp.zeros((pad,), jnp.int32)]).reshape(NW, K_CHUNKS, CHUNK)
    dst_p = jnp.concatenate(
        [dst, jnp.full((pad,), JUNK_ROW, jnp.int32)]).reshape(NW, K_CHUNKS, CHUNK)
    ones_rows = jnp.ones((CHUNK, 16), jnp.float32)
    z16 = jnp.zeros((N_PAD, 16), jnp.float32)
    z64 = jnp.zeros((N_PAD, 64), jnp.float32)
    z32 = jnp.zeros((N_PAD, 32), jnp.float32)

    bins = _deg_call(dst_p, ones_rows, z16)
    u1, dis = _tc1_call(bins[:, :N_NODES, :], x, W1)
    p1 = _prop_call(u1, src_p, dst_p, z64, 64)
    u2 = _tc2_call(p1[:, :N_NODES, :], u1, dis, b1.reshape(1, 64), W2)
    p2 = _prop_call(u2, src_p, dst_p, z32, 32)
    out = _tc3_call(p2[:, :N_NODES, :], u2, dis, b2.reshape(1, 32),
                    batch.astype(jnp.int32).reshape(N_NODES // BLK, 1, BLK))
    return out


# speedup vs baseline: 1.0111x; 1.0111x over previous
import functools

import jax
import jax.numpy as jnp
from jax import lax
from jax.experimental import pallas as pl
from jax.experimental.pallas import tpu as pltpu
from jax.experimental.pallas import tpu_sc as plsc

N_NODES = 10000
NUM_EDGES = 320000
NUM_GRAPHS = 64
NCORE = 2
NSUB = 16
NW = NCORE * NSUB
CHUNK = 128
K_CHUNKS = 80
NBUF = 4
NGRP = K_CHUNKS // NBUF
CAP = NW * K_CHUNKS * CHUNK
N_PAD = 10112
ROWS_PER_TILE = N_PAD // NSUB
JUNK_ROW = 10100
BLK = 1000


def _mesh():
    return plsc.VectorSubcoreMesh(core_axis_name="c", subcore_axis_name="s")


def _deg_call(dst_p, ones_rows, zeros16):
    @functools.partial(
        pl.kernel,
        mesh=_mesh(),
        out_type=jax.ShapeDtypeStruct((NCORE, N_PAD, 16), jnp.float32),
        scratch_types=[
            pltpu.VMEM((K_CHUNKS, CHUNK), jnp.int32),
            pltpu.VMEM((CHUNK, 16), jnp.float32),
            pltpu.VMEM_SHARED((N_PAD, 16), jnp.float32),
            pltpu.SemaphoreType.DMA,
        ],
        compiler_params=pltpu.CompilerParams(use_tc_tiling_on_sc=False),
    )
    def deg_k(dst_hbm, ones_hbm, zeros_hbm, out_hbm, idx_v, ones_v, bins_sh,
              sem):
        c = lax.axis_index("c")
        s = lax.axis_index("s")
        wid = c * NSUB + s
        r0 = pl.multiple_of(s * ROWS_PER_TILE, 8)
        pltpu.sync_copy(zeros_hbm.at[pl.ds(r0, ROWS_PER_TILE)],
                        bins_sh.at[pl.ds(r0, ROWS_PER_TILE)])
        pltpu.sync_copy(ones_hbm, ones_v)
        pltpu.sync_copy(dst_hbm.at[wid], idx_v)
        plsc.subcore_barrier()

        def body(t, carry):
            for b in range(8):
                pltpu.async_copy(ones_v, bins_sh.at[idx_v.at[t * 8 + b]],
                                 sem, add=True)

            @pl.when(t > 0)
            def _():
                for b in range(8):
                    pltpu.make_async_copy(
                        ones_v, bins_sh.at[idx_v.at[b]], sem).wait()

            return carry

        lax.fori_loop(0, K_CHUNKS // 8, body, 0)
        for b in range(8):
            pltpu.make_async_copy(ones_v, bins_sh.at[idx_v.at[b]], sem).wait()
        plsc.subcore_barrier()
        pltpu.sync_copy(bins_sh.at[pl.ds(r0, ROWS_PER_TILE)],
                        out_hbm.at[c, pl.ds(r0, ROWS_PER_TILE)])

    return deg_k(dst_p, ones_rows, zeros16)


def _prop_call(u, src_p, dst_p, zeros, d):
    @functools.partial(
        pl.kernel,
        mesh=_mesh(),
        out_type=jax.ShapeDtypeStruct((NCORE, N_PAD, d), jnp.float32),
        scratch_types=[
            pltpu.VMEM((K_CHUNKS, CHUNK), jnp.int32),
            pltpu.VMEM((K_CHUNKS, CHUNK), jnp.int32),
            pltpu.VMEM((2 * NBUF, CHUNK, d), jnp.float32),
            pltpu.VMEM_SHARED((N_PAD, d), jnp.float32),
        ] + [pltpu.SemaphoreType.DMA] * (4 * NBUF),
        compiler_params=pltpu.CompilerParams(use_tc_tiling_on_sc=False),
    )
    def prop_k(u_hbm, src_hbm, dst_hbm, zeros_hbm, out_hbm,
               src_v, dst_v, rows_v, acc_sh, *sems):
        gsem = sems[:2 * NBUF]
        ssem = sems[2 * NBUF:]
        c = lax.axis_index("c")
        s = lax.axis_index("s")
        wid = c * NSUB + s
        r0 = pl.multiple_of(s * ROWS_PER_TILE, 8)
        pltpu.sync_copy(zeros_hbm.at[pl.ds(r0, ROWS_PER_TILE)],
                        acc_sh.at[pl.ds(r0, ROWS_PER_TILE)])
        pltpu.sync_copy(src_hbm.at[wid], src_v)
        pltpu.sync_copy(dst_hbm.at[wid], dst_v)
        plsc.subcore_barrier()

        def fire_gather(slot, j):
            pltpu.async_copy(u_hbm.at[src_v.at[j]], rows_v.at[slot],
                             gsem[slot])

        def wait_gather(slot, j):
            pltpu.make_async_copy(u_hbm.at[src_v.at[j]], rows_v.at[slot],
                                  gsem[slot]).wait()

        for b in range(2 * NBUF):
            fire_gather(b, b)

        def body(t, carry):
            j0 = 2 * NBUF * t
            for half in range(2):
                off = half * NBUF
                for b in range(NBUF):
                    wait_gather(off + b, j0 + off + b)
                scps = [
                    pltpu.async_copy(rows_v.at[off + b],
                                     acc_sh.at[dst_v.at[j0 + off + b]],
                                     ssem[off + b], add=True)
                    for b in range(NBUF)
                ]
                for d_ in scps:
                    d_.wait()

                @pl.when(t < NGRP // 2 - 1)
                def _():
                    for b in range(NBUF):
                        fire_gather(off + b, j0 + 2 * NBUF + off + b)

            return carry

        lax.fori_loop(0, NGRP // 2, body, 0)
        plsc.subcore_barrier()
        pltpu.sync_copy(acc_sh.at[pl.ds(r0, ROWS_PER_TILE)],
                        out_hbm.at[c, pl.ds(r0, ROWS_PER_TILE)])

    return prop_k(u, src_p, dst_p, zeros)


def _tc1_call(bins, x, W1):
    def body(bins_ref, x_ref, w_ref, u_ref, dis_ref):
        deg = bins_ref[0][:, 0:1] + bins_ref[1][:, 0:1] + 1.0
        dis = lax.rsqrt(deg)
        h = jnp.dot(x_ref[...], w_ref[...], preferred_element_type=jnp.float32)
        u_ref[...] = h * dis
        dis_ref[...] = dis

    return pl.pallas_call(
        body,
        grid=(N_NODES // BLK,),
        in_specs=[
            pl.BlockSpec((NCORE, BLK, 16), lambda i: (0, i, 0)),
            pl.BlockSpec((BLK, 128), lambda i: (i, 0)),
            pl.BlockSpec((128, 64), lambda i: (0, 0)),
        ],
        out_specs=[
            pl.BlockSpec((BLK, 64), lambda i: (i, 0)),
            pl.BlockSpec((BLK, 1), lambda i: (i, 0)),
        ],
        out_shape=[
            jax.ShapeDtypeStruct((N_NODES, 64), jnp.float32),
            jax.ShapeDtypeStruct((N_NODES, 1), jnp.float32),
        ],
    )(bins, x, W1)


def _tc2_call(p, u1, dis, b1, W2):
    def body(p_ref, u1_ref, dis_ref, b1_ref, w2_ref, u2_ref):
        sfull = p_ref[0] + p_ref[1] + u1_ref[...]
        o = jnp.maximum(sfull * dis_ref[...] + b1_ref[...], 0.0)
        u2_ref[...] = jnp.dot(o, w2_ref[...],
                              preferred_element_type=jnp.float32) * dis_ref[...]

    return pl.pallas_call(
        body,
        grid=(N_NODES // BLK,),
        in_specs=[
            pl.BlockSpec((NCORE, BLK, 64), lambda i: (0, i, 0)),
            pl.BlockSpec((BLK, 64), lambda i: (i, 0)),
            pl.BlockSpec((BLK, 1), lambda i: (i, 0)),
            pl.BlockSpec((1, 64), lambda i: (0, 0)),
            pl.BlockSpec((64, 32), lambda i: (0, 0)),
        ],
        out_specs=pl.BlockSpec((BLK, 32), lambda i: (i, 0)),
        out_shape=jax.ShapeDtypeStruct((N_NODES, 32), jnp.float32),
    )(p, u1, dis, b1, W2)


def _tc3_call(p, u2, dis, b2, batch_r):
    def body(p_ref, u2_ref, dis_ref, b2_ref, bt_ref, out_ref):
        h2 = (p_ref[0] + p_ref[1] + u2_ref[...]) * dis_ref[...] + b2_ref[...]
        bt = bt_ref[0]
        oh = (lax.broadcasted_iota(jnp.int32, (NUM_GRAPHS, BLK), 0)
              == bt).astype(jnp.float32)
        acc = jnp.dot(oh, h2, preferred_element_type=jnp.float32)
        i = pl.program_id(0)

        @pl.when(i == 0)
        def _():
            out_ref[...] = acc

        @pl.when(i != 0)
        def _():
            out_ref[...] += acc

    return pl.pallas_call(
        body,
        grid=(N_NODES // BLK,),
        in_specs=[
            pl.BlockSpec((NCORE, BLK, 32), lambda i: (0, i, 0)),
            pl.BlockSpec((BLK, 32), lambda i: (i, 0)),
            pl.BlockSpec((BLK, 1), lambda i: (i, 0)),
            pl.BlockSpec((1, 32), lambda i: (0, 0)),
            pl.BlockSpec((1, 1, BLK), lambda i: (i, 0, 0)),
        ],
        out_specs=pl.BlockSpec((NUM_GRAPHS, 32), lambda i: (0, 0)),
        out_shape=jax.ShapeDtypeStruct((NUM_GRAPHS, 32), jnp.float32),
    )(p, u2, dis, b2, batch_r)


def kernel(x, edge_index, batch, W1, b1, W2, b2):
    src = edge_index[0].astype(jnp.int32)
    dst = edge_index[1].astype(jnp.int32)
    pad = CAP - NUM_EDGES
    src_p = jnp.concatenate(
        [src, jnp.zeros((pad,), jnp.int32)]).reshape(NW, K_CHUNKS, CHUNK)
    pad_dst = N_NODES + (jnp.arange(pad, dtype=jnp.int32) % (N_PAD - N_NODES))
    dst_p = jnp.concatenate([dst, pad_dst]).reshape(NW, K_CHUNKS, CHUNK)
    ones_rows = jnp.ones((CHUNK, 16), jnp.float32)
    z16 = jnp.zeros((N_PAD, 16), jnp.float32)
    z64 = jnp.zeros((N_PAD, 64), jnp.float32)
    z32 = jnp.zeros((N_PAD, 32), jnp.float32)

    bins = _deg_call(dst_p, ones_rows, z16)
    u1, dis = _tc1_call(bins[:, :N_NODES, :], x, W1)
    p1 = _prop_call(u1, src_p, dst_p, z64, 64)
    u2 = _tc2_call(p1[:, :N_NODES, :], u1, dis, b1.reshape(1, 64), W2)
    p2 = _prop_call(u2, src_p, dst_p, z32, 32)
    out = _tc3_call(p2[:, :N_NODES, :], u2, dis, b2.reshape(1, 32),
                    batch.astype(jnp.int32).reshape(N_NODES // BLK, 1, BLK))
    return out


# speedup vs baseline: 1.2101x; 1.1968x over previous
import functools

import jax
import jax.numpy as jnp
from jax import lax
from jax.experimental import pallas as pl
from jax.experimental.pallas import tpu as pltpu
from jax.experimental.pallas import tpu_sc as plsc

N_NODES = 10000
NUM_EDGES = 320000
NUM_GRAPHS = 64
NCORE = 2
NSUB = 16
NW = NCORE * NSUB
CHUNK = 128
K_CHUNKS = 80
NBUF = 4
NGRP = K_CHUNKS // NBUF
CAP = NW * K_CHUNKS * CHUNK
N_PAD = 10112
ROWS_PER_TILE = N_PAD // NSUB
JUNK_ROW = 10100
BLK = 1000


def _mesh():
    return plsc.VectorSubcoreMesh(core_axis_name="c", subcore_axis_name="s")


def _deg_call(dst_p, ones_rows, zeros16):
    @functools.partial(
        pl.kernel,
        mesh=_mesh(),
        out_type=jax.ShapeDtypeStruct((NCORE, N_PAD, 16), jnp.float32),
        scratch_types=[
            pltpu.VMEM((K_CHUNKS, CHUNK), jnp.int32),
            pltpu.VMEM((CHUNK, 16), jnp.float32),
            pltpu.VMEM_SHARED((N_PAD, 16), jnp.float32),
            pltpu.SemaphoreType.DMA,
        ],
        compiler_params=pltpu.CompilerParams(use_tc_tiling_on_sc=False),
    )
    def deg_k(dst_hbm, ones_hbm, zeros_hbm, out_hbm, idx_v, ones_v, bins_sh,
              sem):
        c = lax.axis_index("c")
        s = lax.axis_index("s")
        wid = c * NSUB + s
        r0 = pl.multiple_of(s * ROWS_PER_TILE, 8)
        pltpu.sync_copy(zeros_hbm.at[pl.ds(r0, ROWS_PER_TILE)],
                        bins_sh.at[pl.ds(r0, ROWS_PER_TILE)])
        pltpu.sync_copy(ones_hbm, ones_v)
        pltpu.sync_copy(dst_hbm.at[wid], idx_v)
        plsc.subcore_barrier()

        def body(t, carry):
            for b in range(8):
                pltpu.async_copy(ones_v, bins_sh.at[idx_v.at[t * 8 + b]],
                                 sem, add=True)

            @pl.when(t > 0)
            def _():
                for b in range(8):
                    pltpu.make_async_copy(
                        ones_v, bins_sh.at[idx_v.at[b]], sem).wait()

            return carry

        lax.fori_loop(0, K_CHUNKS // 8, body, 0)
        for b in range(8):
            pltpu.make_async_copy(ones_v, bins_sh.at[idx_v.at[b]], sem).wait()
        plsc.subcore_barrier()
        pltpu.sync_copy(bins_sh.at[pl.ds(r0, ROWS_PER_TILE)],
                        out_hbm.at[c, pl.ds(r0, ROWS_PER_TILE)])

    return deg_k(dst_p, ones_rows, zeros16)


def _prop_call(u, src_p, dst_p, zeros, d, stage_u=False):
    scratch = [
        pltpu.VMEM((K_CHUNKS, CHUNK), jnp.int32),
        pltpu.VMEM((K_CHUNKS, CHUNK), jnp.int32),
        pltpu.VMEM((2 * NBUF, CHUNK, d), jnp.float32),
        pltpu.VMEM_SHARED((N_PAD, d), jnp.float32),
    ]
    if stage_u:
        scratch.append(pltpu.VMEM_SHARED((N_PAD, d), jnp.float32))
    scratch += [pltpu.SemaphoreType.DMA] * (4 * NBUF)

    @functools.partial(
        pl.kernel,
        mesh=_mesh(),
        out_type=jax.ShapeDtypeStruct((NCORE, N_PAD, d), jnp.float32),
        scratch_types=scratch,
        compiler_params=pltpu.CompilerParams(use_tc_tiling_on_sc=False),
    )
    def prop_k(u_hbm, src_hbm, dst_hbm, zeros_hbm, out_hbm,
               src_v, dst_v, rows_v, acc_sh, *rest):
        if stage_u:
            u_src = rest[0]
            sems = rest[1:]
        else:
            u_src = u_hbm
            sems = rest
        gsem = sems[:2 * NBUF]
        ssem = sems[2 * NBUF:]
        c = lax.axis_index("c")
        s = lax.axis_index("s")
        wid = c * NSUB + s
        r0 = pl.multiple_of(s * ROWS_PER_TILE, 8)
        pltpu.sync_copy(zeros_hbm.at[pl.ds(r0, ROWS_PER_TILE)],
                        acc_sh.at[pl.ds(r0, ROWS_PER_TILE)])
        pltpu.sync_copy(src_hbm.at[wid], src_v)
        pltpu.sync_copy(dst_hbm.at[wid], dst_v)
        if stage_u:
            nfull = N_NODES // ROWS_PER_TILE
            rem = N_NODES - nfull * ROWS_PER_TILE

            @pl.when(s < nfull)
            def _():
                pltpu.sync_copy(u_hbm.at[pl.ds(r0, ROWS_PER_TILE)],
                                u_src.at[pl.ds(r0, ROWS_PER_TILE)])

            @pl.when(s == nfull)
            def _():
                rr = pl.multiple_of(nfull * ROWS_PER_TILE, 8)
                pltpu.sync_copy(u_hbm.at[pl.ds(rr, rem)],
                                u_src.at[pl.ds(rr, rem)])

        plsc.subcore_barrier()

        def fire_gather(slot, j):
            pltpu.async_copy(u_src.at[src_v.at[j]], rows_v.at[slot],
                             gsem[slot])

        def wait_gather(slot, j):
            pltpu.make_async_copy(u_src.at[src_v.at[j]], rows_v.at[slot],
                                  gsem[slot]).wait()

        for b in range(2 * NBUF):
            fire_gather(b, b)

        def body(t, carry):
            j0 = 2 * NBUF * t
            for half in range(2):
                off = half * NBUF
                for b in range(NBUF):
                    wait_gather(off + b, j0 + off + b)
                scps = [
                    pltpu.async_copy(rows_v.at[off + b],
                                     acc_sh.at[dst_v.at[j0 + off + b]],
                                     ssem[off + b], add=True)
                    for b in range(NBUF)
                ]
                for d_ in scps:
                    d_.wait()

                @pl.when(t < NGRP // 2 - 1)
                def _():
                    for b in range(NBUF):
                        fire_gather(off + b, j0 + 2 * NBUF + off + b)

            return carry

        lax.fori_loop(0, NGRP // 2, body, 0)
        plsc.subcore_barrier()
        pltpu.sync_copy(acc_sh.at[pl.ds(r0, ROWS_PER_TILE)],
                        out_hbm.at[c, pl.ds(r0, ROWS_PER_TILE)])

    return prop_k(u, src_p, dst_p, zeros)


def _tc1_call(bins, x, W1):
    def body(bins_ref, x_ref, w_ref, u_ref, dis_ref):
        deg = bins_ref[0][:, 0:1] + bins_ref[1][:, 0:1] + 1.0
        dis = lax.rsqrt(deg)
        h = jnp.dot(x_ref[...], w_ref[...], preferred_element_type=jnp.float32)
        u_ref[...] = h * dis
        dis_ref[...] = dis

    return pl.pallas_call(
        body,
        grid=(N_NODES // BLK,),
        in_specs=[
            pl.BlockSpec((NCORE, BLK, 16), lambda i: (0, i, 0)),
            pl.BlockSpec((BLK, 128), lambda i: (i, 0)),
            pl.BlockSpec((128, 64), lambda i: (0, 0)),
        ],
        out_specs=[
            pl.BlockSpec((BLK, 64), lambda i: (i, 0)),
            pl.BlockSpec((BLK, 1), lambda i: (i, 0)),
        ],
        out_shape=[
            jax.ShapeDtypeStruct((N_NODES, 64), jnp.float32),
            jax.ShapeDtypeStruct((N_NODES, 1), jnp.float32),
        ],
    )(bins, x, W1)


def _tc2_call(p, u1, dis, b1, W2):
    def body(p_ref, u1_ref, dis_ref, b1_ref, w2_ref, u2_ref):
        sfull = p_ref[0] + p_ref[1] + u1_ref[...]
        o = jnp.maximum(sfull * dis_ref[...] + b1_ref[...], 0.0)
        u2_ref[...] = jnp.dot(o, w2_ref[...],
                              preferred_element_type=jnp.float32) * dis_ref[...]

    return pl.pallas_call(
        body,
        grid=(N_NODES // BLK,),
        in_specs=[
            pl.BlockSpec((NCORE, BLK, 64), lambda i: (0, i, 0)),
            pl.BlockSpec((BLK, 64), lambda i: (i, 0)),
            pl.BlockSpec((BLK, 1), lambda i: (i, 0)),
            pl.BlockSpec((1, 64), lambda i: (0, 0)),
            pl.BlockSpec((64, 32), lambda i: (0, 0)),
        ],
        out_specs=pl.BlockSpec((BLK, 32), lambda i: (i, 0)),
        out_shape=jax.ShapeDtypeStruct((N_NODES, 32), jnp.float32),
    )(p, u1, dis, b1, W2)


def _tc3_call(p, u2, dis, b2, batch_r):
    def body(p_ref, u2_ref, dis_ref, b2_ref, bt_ref, out_ref):
        h2 = (p_ref[0] + p_ref[1] + u2_ref[...]) * dis_ref[...] + b2_ref[...]
        bt = bt_ref[0]
        oh = (lax.broadcasted_iota(jnp.int32, (NUM_GRAPHS, BLK), 0)
              == bt).astype(jnp.float32)
        acc = jnp.dot(oh, h2, preferred_element_type=jnp.float32)
        i = pl.program_id(0)

        @pl.when(i == 0)
        def _():
            out_ref[...] = acc

        @pl.when(i != 0)
        def _():
            out_ref[...] += acc

    return pl.pallas_call(
        body,
        grid=(N_NODES // BLK,),
        in_specs=[
            pl.BlockSpec((NCORE, BLK, 32), lambda i: (0, i, 0)),
            pl.BlockSpec((BLK, 32), lambda i: (i, 0)),
            pl.BlockSpec((BLK, 1), lambda i: (i, 0)),
            pl.BlockSpec((1, 32), lambda i: (0, 0)),
            pl.BlockSpec((1, 1, BLK), lambda i: (i, 0, 0)),
        ],
        out_specs=pl.BlockSpec((NUM_GRAPHS, 32), lambda i: (0, 0)),
        out_shape=jax.ShapeDtypeStruct((NUM_GRAPHS, 32), jnp.float32),
    )(p, u2, dis, b2, batch_r)


def kernel(x, edge_index, batch, W1, b1, W2, b2):
    src = edge_index[0].astype(jnp.int32)
    dst = edge_index[1].astype(jnp.int32)
    pad = CAP - NUM_EDGES
    src_p = jnp.concatenate(
        [src, jnp.zeros((pad,), jnp.int32)]).reshape(NW, K_CHUNKS, CHUNK)
    pad_dst = N_NODES + (jnp.arange(pad, dtype=jnp.int32) % (N_PAD - N_NODES))
    dst_p = jnp.concatenate([dst, pad_dst]).reshape(NW, K_CHUNKS, CHUNK)
    ones_rows = jnp.ones((CHUNK, 16), jnp.float32)
    z16 = jnp.zeros((N_PAD, 16), jnp.float32)
    z64 = jnp.zeros((N_PAD, 64), jnp.float32)
    z32 = jnp.zeros((N_PAD, 32), jnp.float32)

    bins = _deg_call(dst_p, ones_rows, z16)
    u1, dis = _tc1_call(bins[:, :N_NODES, :], x, W1)
    p1 = _prop_call(u1, src_p, dst_p, z64, 64)
    u2 = _tc2_call(p1[:, :N_NODES, :], u1, dis, b1.reshape(1, 64), W2)
    p2 = _prop_call(u2, src_p, dst_p, z32, 32, stage_u=True)
    out = _tc3_call(p2[:, :N_NODES, :], u2, dis, b2.reshape(1, 32),
                    batch.astype(jnp.int32).reshape(N_NODES // BLK, 1, BLK))
    return out


# speedup vs baseline: 1.8472x; 1.5265x over previous
import functools

import jax
import jax.numpy as jnp
from jax import lax
from jax.experimental import pallas as pl
from jax.experimental.pallas import tpu as pltpu
from jax.experimental.pallas import tpu_sc as plsc

N_NODES = 10000
NUM_EDGES = 320000
NUM_GRAPHS = 64
NCORE = 2
NSUB = 16
NW = NCORE * NSUB
CHUNK = 128
K_CHUNKS = 80
NBUF = 4
NGRP = K_CHUNKS // NBUF
CAP = NW * K_CHUNKS * CHUNK
N_PAD = 10112
ROWS_PER_TILE = N_PAD // NSUB
JUNK_ROW = 10100
BLK = 1000


def _mesh():
    return plsc.VectorSubcoreMesh(core_axis_name="c", subcore_axis_name="s")


def _deg_call(dst_p, ones_rows, zeros16):
    @functools.partial(
        pl.kernel,
        mesh=_mesh(),
        out_type=jax.ShapeDtypeStruct((NCORE, N_PAD, 16), jnp.float32),
        scratch_types=[
            pltpu.VMEM((K_CHUNKS, CHUNK), jnp.int32),
            pltpu.VMEM((CHUNK, 16), jnp.float32),
            pltpu.VMEM_SHARED((N_PAD, 16), jnp.float32),
            pltpu.SemaphoreType.DMA,
        ],
        compiler_params=pltpu.CompilerParams(use_tc_tiling_on_sc=False),
    )
    def deg_k(dst_hbm, ones_hbm, zeros_hbm, out_hbm, idx_v, ones_v, bins_sh,
              sem):
        c = lax.axis_index("c")
        s = lax.axis_index("s")
        wid = c * NSUB + s
        r0 = pl.multiple_of(s * ROWS_PER_TILE, 8)
        pltpu.sync_copy(zeros_hbm.at[pl.ds(r0, ROWS_PER_TILE)],
                        bins_sh.at[pl.ds(r0, ROWS_PER_TILE)])
        pltpu.sync_copy(ones_hbm, ones_v)
        pltpu.sync_copy(dst_hbm.at[wid], idx_v)
        plsc.subcore_barrier()

        def body(t, carry):
            for b in range(8):
                pltpu.async_copy(ones_v, bins_sh.at[idx_v.at[t * 8 + b]],
                                 sem, add=True)

            @pl.when(t > 0)
            def _():
                for b in range(8):
                    pltpu.make_async_copy(
                        ones_v, bins_sh.at[idx_v.at[b]], sem).wait()

            return carry

        lax.fori_loop(0, K_CHUNKS // 8, body, 0)
        for b in range(8):
            pltpu.make_async_copy(ones_v, bins_sh.at[idx_v.at[b]], sem).wait()
        plsc.subcore_barrier()
        pltpu.sync_copy(bins_sh.at[pl.ds(r0, ROWS_PER_TILE)],
                        out_hbm.at[c, pl.ds(r0, ROWS_PER_TILE)])

    return deg_k(dst_p, ones_rows, zeros16)


def _prop_call(u, src_p, dst_p, zeros, d, stage_u=False):
    scratch = [
        pltpu.VMEM((K_CHUNKS, CHUNK), jnp.int32),
        pltpu.VMEM((K_CHUNKS, CHUNK), jnp.int32),
        pltpu.VMEM((2 * NBUF, CHUNK, d), jnp.float32),
        pltpu.VMEM_SHARED((N_PAD, d), jnp.float32),
    ]
    if stage_u:
        scratch.append(pltpu.VMEM_SHARED((N_PAD, d), jnp.float32))
    scratch += [pltpu.SemaphoreType.DMA] * (4 * NBUF)

    @functools.partial(
        pl.kernel,
        mesh=_mesh(),
        out_type=jax.ShapeDtypeStruct((NCORE, N_PAD, d), jnp.float32),
        scratch_types=scratch,
        compiler_params=pltpu.CompilerParams(use_tc_tiling_on_sc=False),
    )
    def prop_k(u_hbm, src_hbm, dst_hbm, zeros_hbm, out_hbm,
               src_v, dst_v, rows_v, acc_sh, *rest):
        if stage_u:
            u_src = rest[0]
            sems = rest[1:]
        else:
            u_src = u_hbm
            sems = rest
        gsem = sems[:2 * NBUF]
        ssem = sems[2 * NBUF:]
        c = lax.axis_index("c")
        s = lax.axis_index("s")
        wid = c * NSUB + s
        r0 = pl.multiple_of(s * ROWS_PER_TILE, 8)
        pltpu.sync_copy(zeros_hbm.at[pl.ds(r0, ROWS_PER_TILE)],
                        acc_sh.at[pl.ds(r0, ROWS_PER_TILE)])
        pltpu.sync_copy(src_hbm.at[wid], src_v)
        pltpu.sync_copy(dst_hbm.at[wid], dst_v)
        if stage_u:
            nfull = N_NODES // ROWS_PER_TILE
            rem = N_NODES - nfull * ROWS_PER_TILE

            @pl.when(s < nfull)
            def _():
                pltpu.sync_copy(u_hbm.at[pl.ds(r0, ROWS_PER_TILE)],
                                u_src.at[pl.ds(r0, ROWS_PER_TILE)])

            @pl.when(s == nfull)
            def _():
                rr = pl.multiple_of(nfull * ROWS_PER_TILE, 8)
                pltpu.sync_copy(u_hbm.at[pl.ds(rr, rem)],
                                u_src.at[pl.ds(rr, rem)])

        plsc.subcore_barrier()

        def fire_gather(slot, j):
            pltpu.async_copy(u_src.at[src_v.at[j]], rows_v.at[slot],
                             gsem[slot])

        def wait_gather(slot, j):
            pltpu.make_async_copy(u_src.at[src_v.at[j]], rows_v.at[slot],
                                  gsem[slot]).wait()

        for b in range(2 * NBUF):
            fire_gather(b, b)

        def body(t, carry):
            j0 = 2 * NBUF * t
            for half in range(2):
                off = half * NBUF
                for b in range(NBUF):
                    wait_gather(off + b, j0 + off + b)
                scps = [
                    pltpu.async_copy(rows_v.at[off + b],
                                     acc_sh.at[dst_v.at[j0 + off + b]],
                                     ssem[off + b], add=True)
                    for b in range(NBUF)
                ]
                for d_ in scps:
                    d_.wait()

                @pl.when(t < NGRP // 2 - 1)
                def _():
                    for b in range(NBUF):
                        fire_gather(off + b, j0 + 2 * NBUF + off + b)

            return carry

        lax.fori_loop(0, NGRP // 2, body, 0)
        plsc.subcore_barrier()
        pltpu.sync_copy(acc_sh.at[pl.ds(r0, ROWS_PER_TILE)],
                        out_hbm.at[c, pl.ds(r0, ROWS_PER_TILE)])

    return prop_k(u, src_p, dst_p, zeros)


def _tc1_call(bins, x, W1):
    def body(bins_ref, x_ref, w_ref, ua_ref, ub_ref, dis_ref):
        deg = bins_ref[0][:, 0:1] + bins_ref[1][:, 0:1] + 1.0
        dis = lax.rsqrt(deg)
        h = jnp.dot(x_ref[...], w_ref[...], preferred_element_type=jnp.float32)
        u = h * dis
        ua_ref[...] = u[:, :32]
        ub_ref[...] = u[:, 32:]
        dis_ref[...] = dis

    return pl.pallas_call(
        body,
        grid=(N_NODES // BLK,),
        in_specs=[
            pl.BlockSpec((NCORE, BLK, 16), lambda i: (0, i, 0)),
            pl.BlockSpec((BLK, 128), lambda i: (i, 0)),
            pl.BlockSpec((128, 64), lambda i: (0, 0)),
        ],
        out_specs=[
            pl.BlockSpec((BLK, 32), lambda i: (i, 0)),
            pl.BlockSpec((BLK, 32), lambda i: (i, 0)),
            pl.BlockSpec((BLK, 1), lambda i: (i, 0)),
        ],
        out_shape=[
            jax.ShapeDtypeStruct((N_NODES, 32), jnp.float32),
            jax.ShapeDtypeStruct((N_NODES, 32), jnp.float32),
            jax.ShapeDtypeStruct((N_NODES, 1), jnp.float32),
        ],
    )(bins, x, W1)


def _tc2_call(pa, pb, u1a, u1b, dis, b1, W2):
    def body(pa_ref, pb_ref, ua_ref, ub_ref, dis_ref, b1_ref, w2_ref, u2_ref):
        sa = pa_ref[0] + pa_ref[1] + ua_ref[...]
        sb = pb_ref[0] + pb_ref[1] + ub_ref[...]
        sfull = jnp.concatenate([sa, sb], axis=-1)
        o = jnp.maximum(sfull * dis_ref[...] + b1_ref[...], 0.0)
        u2_ref[...] = jnp.dot(o, w2_ref[...],
                              preferred_element_type=jnp.float32) * dis_ref[...]

    return pl.pallas_call(
        body,
        grid=(N_NODES // BLK,),
        in_specs=[
            pl.BlockSpec((NCORE, BLK, 32), lambda i: (0, i, 0)),
            pl.BlockSpec((NCORE, BLK, 32), lambda i: (0, i, 0)),
            pl.BlockSpec((BLK, 32), lambda i: (i, 0)),
            pl.BlockSpec((BLK, 32), lambda i: (i, 0)),
            pl.BlockSpec((BLK, 1), lambda i: (i, 0)),
            pl.BlockSpec((1, 64), lambda i: (0, 0)),
            pl.BlockSpec((64, 32), lambda i: (0, 0)),
        ],
        out_specs=pl.BlockSpec((BLK, 32), lambda i: (i, 0)),
        out_shape=jax.ShapeDtypeStruct((N_NODES, 32), jnp.float32),
    )(pa, pb, u1a, u1b, dis, b1, W2)


def _tc3_call(p, u2, dis, b2, batch_r):
    def body(p_ref, u2_ref, dis_ref, b2_ref, bt_ref, out_ref):
        h2 = (p_ref[0] + p_ref[1] + u2_ref[...]) * dis_ref[...] + b2_ref[...]
        bt = bt_ref[0]
        oh = (lax.broadcasted_iota(jnp.int32, (NUM_GRAPHS, BLK), 0)
              == bt).astype(jnp.float32)
        acc = jnp.dot(oh, h2, preferred_element_type=jnp.float32)
        i = pl.program_id(0)

        @pl.when(i == 0)
        def _():
            out_ref[...] = acc

        @pl.when(i != 0)
        def _():
            out_ref[...] += acc

    return pl.pallas_call(
        body,
        grid=(N_NODES // BLK,),
        in_specs=[
            pl.BlockSpec((NCORE, BLK, 32), lambda i: (0, i, 0)),
            pl.BlockSpec((BLK, 32), lambda i: (i, 0)),
            pl.BlockSpec((BLK, 1), lambda i: (i, 0)),
            pl.BlockSpec((1, 32), lambda i: (0, 0)),
            pl.BlockSpec((1, 1, BLK), lambda i: (i, 0, 0)),
        ],
        out_specs=pl.BlockSpec((NUM_GRAPHS, 32), lambda i: (0, 0)),
        out_shape=jax.ShapeDtypeStruct((NUM_GRAPHS, 32), jnp.float32),
    )(p, u2, dis, b2, batch_r)


def kernel(x, edge_index, batch, W1, b1, W2, b2):
    src = edge_index[0].astype(jnp.int32)
    dst = edge_index[1].astype(jnp.int32)
    pad = CAP - NUM_EDGES
    src_p = jnp.concatenate(
        [src, jnp.zeros((pad,), jnp.int32)]).reshape(NW, K_CHUNKS, CHUNK)
    pad_dst = N_NODES + (jnp.arange(pad, dtype=jnp.int32) % (N_PAD - N_NODES))
    dst_p = jnp.concatenate([dst, pad_dst]).reshape(NW, K_CHUNKS, CHUNK)
    ones_rows = jnp.ones((CHUNK, 16), jnp.float32)
    z16 = jnp.zeros((N_PAD, 16), jnp.float32)
    z32 = jnp.zeros((N_PAD, 32), jnp.float32)

    bins = _deg_call(dst_p, ones_rows, z16)
    u1a, u1b, dis = _tc1_call(bins[:, :N_NODES, :], x, W1)
    p1a = _prop_call(u1a, src_p, dst_p, z32, 32, stage_u=True)
    p1b = _prop_call(u1b, src_p, dst_p, z32, 32, stage_u=True)
    u2 = _tc2_call(p1a[:, :N_NODES, :], p1b[:, :N_NODES, :], u1a, u1b, dis,
                   b1.reshape(1, 64), W2)
    p2 = _prop_call(u2, src_p, dst_p, z32, 32, stage_u=True)
    out = _tc3_call(p2[:, :N_NODES, :], u2, dis, b2.reshape(1, 32),
                    batch.astype(jnp.int32).reshape(N_NODES // BLK, 1, BLK))
    return out


# speedup vs baseline: 2.0668x; 1.1189x over previous
import functools

import jax
import jax.numpy as jnp
from jax import lax
from jax.experimental import pallas as pl
from jax.experimental.pallas import tpu as pltpu
from jax.experimental.pallas import tpu_sc as plsc

N_NODES = 10000
NUM_EDGES = 320000
NUM_GRAPHS = 64
NCORE = 2
NSUB = 16
NW = NCORE * NSUB
CHUNK = 128
K_CHUNKS = 80
NBUF = 4
NGRP = K_CHUNKS // NBUF
CAP = NW * K_CHUNKS * CHUNK
N_PAD = 10112
ROWS_PER_TILE = N_PAD // NSUB
JUNK_ROW = 10100
BLK = 1000


def _mesh():
    return plsc.VectorSubcoreMesh(core_axis_name="c", subcore_axis_name="s")


def _deg_call(ei_p, ones_rows, zeros16):
    @functools.partial(
        pl.kernel,
        mesh=_mesh(),
        out_type=jax.ShapeDtypeStruct((NCORE, N_PAD, 16), jnp.float32),
        scratch_types=[
            pltpu.VMEM((K_CHUNKS, CHUNK), jnp.int32),
            pltpu.VMEM((CHUNK, 16), jnp.float32),
            pltpu.VMEM_SHARED((N_PAD, 16), jnp.float32),
            pltpu.SemaphoreType.DMA,
        ],
        compiler_params=pltpu.CompilerParams(use_tc_tiling_on_sc=False),
    )
    def deg_k(ei_hbm, ones_hbm, zeros_hbm, out_hbm, idx_v, ones_v, bins_sh,
              sem):
        c = lax.axis_index("c")
        s = lax.axis_index("s")
        wid = c * NSUB + s
        r0 = pl.multiple_of(s * ROWS_PER_TILE, 8)
        pltpu.sync_copy(zeros_hbm.at[pl.ds(r0, ROWS_PER_TILE)],
                        bins_sh.at[pl.ds(r0, ROWS_PER_TILE)])
        pltpu.sync_copy(ones_hbm, ones_v)
        pltpu.sync_copy(ei_hbm.at[1, wid], idx_v)
        plsc.subcore_barrier()

        def body(t, carry):
            for b in range(8):
                pltpu.async_copy(ones_v, bins_sh.at[idx_v.at[t * 8 + b]],
                                 sem, add=True)

            @pl.when(t > 0)
            def _():
                for b in range(8):
                    pltpu.make_async_copy(
                        ones_v, bins_sh.at[idx_v.at[b]], sem).wait()

            return carry

        lax.fori_loop(0, K_CHUNKS // 8, body, 0)
        for b in range(8):
            pltpu.make_async_copy(ones_v, bins_sh.at[idx_v.at[b]], sem).wait()
        plsc.subcore_barrier()
        pltpu.sync_copy(bins_sh.at[pl.ds(r0, ROWS_PER_TILE)],
                        out_hbm.at[c, pl.ds(r0, ROWS_PER_TILE)])

    return deg_k(ei_p, ones_rows, zeros16)


def _prop_call(u, ei_p, zeros, d, stage_u=False):
    scratch = [
        pltpu.VMEM((K_CHUNKS, CHUNK), jnp.int32),
        pltpu.VMEM((K_CHUNKS, CHUNK), jnp.int32),
        pltpu.VMEM((2 * NBUF, CHUNK, d), jnp.float32),
        pltpu.VMEM_SHARED((N_PAD, d), jnp.float32),
    ]
    if stage_u:
        scratch.append(pltpu.VMEM_SHARED((N_PAD, d), jnp.float32))
    scratch += [pltpu.SemaphoreType.DMA] * (4 * NBUF)

    @functools.partial(
        pl.kernel,
        mesh=_mesh(),
        out_type=jax.ShapeDtypeStruct((NCORE, N_PAD, d), jnp.float32),
        scratch_types=scratch,
        compiler_params=pltpu.CompilerParams(use_tc_tiling_on_sc=False),
    )
    def prop_k(u_hbm, ei_hbm, zeros_hbm, out_hbm,
               src_v, dst_v, rows_v, acc_sh, *rest):
        if stage_u:
            u_src = rest[0]
            sems = rest[1:]
        else:
            u_src = u_hbm
            sems = rest
        gsem = sems[:2 * NBUF]
        ssem = sems[2 * NBUF:]
        c = lax.axis_index("c")
        s = lax.axis_index("s")
        wid = c * NSUB + s
        r0 = pl.multiple_of(s * ROWS_PER_TILE, 8)
        pltpu.sync_copy(zeros_hbm.at[pl.ds(r0, ROWS_PER_TILE)],
                        acc_sh.at[pl.ds(r0, ROWS_PER_TILE)])
        pltpu.sync_copy(ei_hbm.at[0, wid], src_v)
        pltpu.sync_copy(ei_hbm.at[1, wid], dst_v)
        if stage_u:
            nfull = N_NODES // ROWS_PER_TILE
            rem = N_NODES - nfull * ROWS_PER_TILE

            @pl.when(s < nfull)
            def _():
                pltpu.sync_copy(u_hbm.at[pl.ds(r0, ROWS_PER_TILE)],
                                u_src.at[pl.ds(r0, ROWS_PER_TILE)])

            @pl.when(s == nfull)
            def _():
                rr = pl.multiple_of(nfull * ROWS_PER_TILE, 8)
                pltpu.sync_copy(u_hbm.at[pl.ds(rr, rem)],
                                u_src.at[pl.ds(rr, rem)])

        plsc.subcore_barrier()

        def fire_gather(slot, j):
            pltpu.async_copy(u_src.at[src_v.at[j]], rows_v.at[slot],
                             gsem[slot])

        def wait_gather(slot, j):
            pltpu.make_async_copy(u_src.at[src_v.at[j]], rows_v.at[slot],
                                  gsem[slot]).wait()

        for b in range(2 * NBUF):
            fire_gather(b, b)

        def body(t, carry):
            j0 = 2 * NBUF * t
            for half in range(2):
                off = half * NBUF
                for b in range(NBUF):
                    wait_gather(off + b, j0 + off + b)
                scps = [
                    pltpu.async_copy(rows_v.at[off + b],
                                     acc_sh.at[dst_v.at[j0 + off + b]],
                                     ssem[off + b], add=True)
                    for b in range(NBUF)
                ]
                for d_ in scps:
                    d_.wait()

                @pl.when(t < NGRP // 2 - 1)
                def _():
                    for b in range(NBUF):
                        fire_gather(off + b, j0 + 2 * NBUF + off + b)

            return carry

        lax.fori_loop(0, NGRP // 2, body, 0)
        plsc.subcore_barrier()
        pltpu.sync_copy(acc_sh.at[pl.ds(r0, ROWS_PER_TILE)],
                        out_hbm.at[c, pl.ds(r0, ROWS_PER_TILE)])

    return prop_k(u, ei_p, zeros)


def _tc1_call(bins, x, W1):
    def body(bins_ref, x_ref, w_ref, ua_ref, ub_ref, dis_ref):
        deg = bins_ref[0][:, 0:1] + bins_ref[1][:, 0:1] + 1.0
        dis = lax.rsqrt(deg)
        h = jnp.dot(x_ref[...], w_ref[...], preferred_element_type=jnp.float32)
        u = h * dis
        ua_ref[...] = u[:, :32]
        ub_ref[...] = u[:, 32:]
        dis_ref[...] = dis

    return pl.pallas_call(
        body,
        grid=(N_NODES // BLK,),
        in_specs=[
            pl.BlockSpec((NCORE, BLK, 16), lambda i: (0, i, 0)),
            pl.BlockSpec((BLK, 128), lambda i: (i, 0)),
            pl.BlockSpec((128, 64), lambda i: (0, 0)),
        ],
        out_specs=[
            pl.BlockSpec((BLK, 32), lambda i: (i, 0)),
            pl.BlockSpec((BLK, 32), lambda i: (i, 0)),
            pl.BlockSpec((BLK, 1), lambda i: (i, 0)),
        ],
        out_shape=[
            jax.ShapeDtypeStruct((N_NODES, 32), jnp.float32),
            jax.ShapeDtypeStruct((N_NODES, 32), jnp.float32),
            jax.ShapeDtypeStruct((N_NODES, 1), jnp.float32),
        ],
    )(bins, x, W1)


def _tc2_call(pa, pb, u1a, u1b, dis, b1, W2):
    def body(pa_ref, pb_ref, ua_ref, ub_ref, dis_ref, b1_ref, w2_ref, u2_ref):
        sa = pa_ref[0] + pa_ref[1] + ua_ref[...]
        sb = pb_ref[0] + pb_ref[1] + ub_ref[...]
        sfull = jnp.concatenate([sa, sb], axis=-1)
        o = jnp.maximum(sfull * dis_ref[...] + b1_ref[...], 0.0)
        u2_ref[...] = jnp.dot(o, w2_ref[...],
                              preferred_element_type=jnp.float32) * dis_ref[...]

    return pl.pallas_call(
        body,
        grid=(N_NODES // BLK,),
        in_specs=[
            pl.BlockSpec((NCORE, BLK, 32), lambda i: (0, i, 0)),
            pl.BlockSpec((NCORE, BLK, 32), lambda i: (0, i, 0)),
            pl.BlockSpec((BLK, 32), lambda i: (i, 0)),
            pl.BlockSpec((BLK, 32), lambda i: (i, 0)),
            pl.BlockSpec((BLK, 1), lambda i: (i, 0)),
            pl.BlockSpec((1, 64), lambda i: (0, 0)),
            pl.BlockSpec((64, 32), lambda i: (0, 0)),
        ],
        out_specs=pl.BlockSpec((BLK, 32), lambda i: (i, 0)),
        out_shape=jax.ShapeDtypeStruct((N_NODES, 32), jnp.float32),
    )(pa, pb, u1a, u1b, dis, b1, W2)


def _tc3_call(p, u2, dis, b2, batch_r):
    def body(p_ref, u2_ref, dis_ref, b2_ref, bt_ref, out_ref):
        h2 = (p_ref[0] + p_ref[1] + u2_ref[...]) * dis_ref[...] + b2_ref[...]
        bt = bt_ref[0]
        oh = (lax.broadcasted_iota(jnp.int32, (NUM_GRAPHS, BLK), 0)
              == bt).astype(jnp.float32)
        acc = jnp.dot(oh, h2, preferred_element_type=jnp.float32)
        i = pl.program_id(0)

        @pl.when(i == 0)
        def _():
            out_ref[...] = acc

        @pl.when(i != 0)
        def _():
            out_ref[...] += acc

    return pl.pallas_call(
        body,
        grid=(N_NODES // BLK,),
        in_specs=[
            pl.BlockSpec((NCORE, BLK, 32), lambda i: (0, i, 0)),
            pl.BlockSpec((BLK, 32), lambda i: (i, 0)),
            pl.BlockSpec((BLK, 1), lambda i: (i, 0)),
            pl.BlockSpec((1, 32), lambda i: (0, 0)),
            pl.BlockSpec((1, 1, BLK), lambda i: (i, 0, 0)),
        ],
        out_specs=pl.BlockSpec((NUM_GRAPHS, 32), lambda i: (0, 0)),
        out_shape=jax.ShapeDtypeStruct((NUM_GRAPHS, 32), jnp.float32),
    )(p, u2, dis, b2, batch_r)


def kernel(x, edge_index, batch, W1, b1, W2, b2):
    pad = CAP - NUM_EDGES
    pad_src = jnp.zeros((pad,), jnp.int32)
    pad_dst = N_NODES + (jnp.arange(pad, dtype=jnp.int32) % (N_PAD - N_NODES))
    ei_p = jnp.concatenate(
        [edge_index.astype(jnp.int32), jnp.stack([pad_src, pad_dst])],
        axis=1).reshape(2, NW, K_CHUNKS, CHUNK)
    ones_rows = jnp.ones((CHUNK, 16), jnp.float32)
    z16 = jnp.zeros((N_PAD, 16), jnp.float32)
    z32 = jnp.zeros((N_PAD, 32), jnp.float32)

    bins = _deg_call(ei_p, ones_rows, z16)
    u1a, u1b, dis = _tc1_call(bins, x, W1)
    p1a = _prop_call(u1a, ei_p, z32, 32, stage_u=True)
    p1b = _prop_call(u1b, ei_p, z32, 32, stage_u=True)
    u2 = _tc2_call(p1a, p1b, u1a, u1b, dis, b1.reshape(1, 64), W2)
    p2 = _prop_call(u2, ei_p, z32, 32, stage_u=True)
    out = _tc3_call(p2, u2, dis, b2.reshape(1, 32),
                    batch.astype(jnp.int32).reshape(N_NODES // BLK, 1, BLK))
    return out


# speedup vs baseline: 2.2576x; 1.0924x over previous
import functools

import jax
import jax.numpy as jnp
from jax import lax
from jax.experimental import pallas as pl
from jax.experimental.pallas import tpu as pltpu
from jax.experimental.pallas import tpu_sc as plsc

N_NODES = 10000
NUM_EDGES = 320000
NUM_GRAPHS = 64
NCORE = 2
NSUB = 16
NW = NCORE * NSUB
CHUNK = 128
N_CHUNKS = 2560
NBUF = 4
CAP = N_CHUNKS * CHUNK
N_PAD = 10112
ROWS_PER_TILE = N_PAD // NSUB
BLK = 1000


def _mesh():
    return plsc.VectorSubcoreMesh(core_axis_name="c", subcore_axis_name="s")


def _stage_rows(u_hbm, u_sh, s):
    r0 = pl.multiple_of(s * ROWS_PER_TILE, 8)
    nfull = N_NODES // ROWS_PER_TILE
    rem = N_NODES - nfull * ROWS_PER_TILE

    @pl.when(s < nfull)
    def _():
        pltpu.sync_copy(u_hbm.at[pl.ds(r0, ROWS_PER_TILE)],
                        u_sh.at[pl.ds(r0, ROWS_PER_TILE)])

    @pl.when(s == nfull)
    def _():
        rr = pl.multiple_of(nfull * ROWS_PER_TILE, 8)
        pltpu.sync_copy(u_hbm.at[pl.ds(rr, rem)], u_sh.at[pl.ds(rr, rem)])


def _gather_scatter_loop(u_src, acc_sh, src_v, dst_v, rows_v, gsem, ssem,
                         k_tile):
    nstep = k_tile // (2 * NBUF)

    def fire_gather(slot, j):
        pltpu.async_copy(u_src.at[src_v.at[j]], rows_v.at[slot], gsem[slot])

    def wait_gather(slot, j):
        pltpu.make_async_copy(u_src.at[src_v.at[j]], rows_v.at[slot],
                              gsem[slot]).wait()

    for b in range(2 * NBUF):
        fire_gather(b, b)

    def body(t, carry):
        j0 = 2 * NBUF * t
        for half in range(2):
            off = half * NBUF
            for b in range(NBUF):
                wait_gather(off + b, j0 + off + b)
            scps = [
                pltpu.async_copy(rows_v.at[off + b],
                                 acc_sh.at[dst_v.at[j0 + off + b]],
                                 ssem[off + b], add=True)
                for b in range(NBUF)
            ]
            for d_ in scps:
                d_.wait()

            @pl.when(t < nstep - 1)
            def _():
                for b in range(NBUF):
                    fire_gather(off + b, j0 + 2 * NBUF + off + b)

        return carry

    lax.fori_loop(0, nstep, body, 0)


def _deg_call(ei_p, ones_rows, zeros16):
    k_tile = N_CHUNKS // NW

    @functools.partial(
        pl.kernel,
        mesh=_mesh(),
        out_type=jax.ShapeDtypeStruct((NCORE, N_PAD, 16), jnp.float32),
        scratch_types=[
            pltpu.VMEM((k_tile, CHUNK), jnp.int32),
            pltpu.VMEM((CHUNK, 16), jnp.float32),
            pltpu.VMEM_SHARED((N_PAD, 16), jnp.float32),
            pltpu.SemaphoreType.DMA,
        ],
        compiler_params=pltpu.CompilerParams(use_tc_tiling_on_sc=False),
    )
    def deg_k(ei_hbm, ones_hbm, zeros_hbm, out_hbm, idx_v, ones_v, bins_sh,
              sem):
        c = lax.axis_index("c")
        s = lax.axis_index("s")
        wid = c * NSUB + s
        r0 = pl.multiple_of(s * ROWS_PER_TILE, 8)
        pltpu.sync_copy(zeros_hbm.at[pl.ds(r0, ROWS_PER_TILE)],
                        bins_sh.at[pl.ds(r0, ROWS_PER_TILE)])
        pltpu.sync_copy(ones_hbm, ones_v)
        pltpu.sync_copy(ei_hbm.at[1, pl.ds(wid * k_tile, k_tile)], idx_v)
        plsc.subcore_barrier()

        def body(t, carry):
            for b in range(8):
                pltpu.async_copy(ones_v, bins_sh.at[idx_v.at[t * 8 + b]],
                                 sem, add=True)

            @pl.when(t > 0)
            def _():
                for b in range(8):
                    pltpu.make_async_copy(
                        ones_v, bins_sh.at[idx_v.at[b]], sem).wait()

            return carry

        lax.fori_loop(0, k_tile // 8, body, 0)
        for b in range(8):
            pltpu.make_async_copy(ones_v, bins_sh.at[idx_v.at[b]], sem).wait()
        plsc.subcore_barrier()
        pltpu.sync_copy(bins_sh.at[pl.ds(r0, ROWS_PER_TILE)],
                        out_hbm.at[c, pl.ds(r0, ROWS_PER_TILE)])

    return deg_k(ei_p, ones_rows, zeros16)


def _prop1_call(ua, ub, ei_p, zeros):
    k_tile = N_CHUNKS // NSUB

    @functools.partial(
        pl.kernel,
        mesh=_mesh(),
        out_type=jax.ShapeDtypeStruct((NCORE, N_PAD, 32), jnp.float32),
        scratch_types=[
            pltpu.VMEM((k_tile, CHUNK), jnp.int32),
            pltpu.VMEM((k_tile, CHUNK), jnp.int32),
            pltpu.VMEM((2 * NBUF, CHUNK, 32), jnp.float32),
            pltpu.VMEM_SHARED((N_PAD, 32), jnp.float32),
            pltpu.VMEM_SHARED((N_PAD, 32), jnp.float32),
        ] + [pltpu.SemaphoreType.DMA] * (4 * NBUF),
        compiler_params=pltpu.CompilerParams(use_tc_tiling_on_sc=False),
    )
    def prop1_k(ua_hbm, ub_hbm, ei_hbm, zeros_hbm, out_hbm,
                src_v, dst_v, rows_v, acc_sh, u_sh, *sems):
        gsem = sems[:2 * NBUF]
        ssem = sems[2 * NBUF:]
        c = lax.axis_index("c")
        s = lax.axis_index("s")
        r0 = pl.multiple_of(s * ROWS_PER_TILE, 8)
        pltpu.sync_copy(zeros_hbm.at[pl.ds(r0, ROWS_PER_TILE)],
                        acc_sh.at[pl.ds(r0, ROWS_PER_TILE)])
        pltpu.sync_copy(ei_hbm.at[0, pl.ds(s * k_tile, k_tile)], src_v)
        pltpu.sync_copy(ei_hbm.at[1, pl.ds(s * k_tile, k_tile)], dst_v)

        @pl.when(c == 0)
        def _():
            _stage_rows(ua_hbm, u_sh, s)

        @pl.when(c == 1)
        def _():
            _stage_rows(ub_hbm, u_sh, s)

        plsc.subcore_barrier()
        _gather_scatter_loop(u_sh, acc_sh, src_v, dst_v, rows_v, gsem, ssem,
                             k_tile)
        plsc.subcore_barrier()
        pltpu.sync_copy(acc_sh.at[pl.ds(r0, ROWS_PER_TILE)],
                        out_hbm.at[c, pl.ds(r0, ROWS_PER_TILE)])

    return prop1_k(ua, ub, ei_p, zeros)


def _prop2_call(u, ei_p, zeros):
    k_tile = N_CHUNKS // NW

    @functools.partial(
        pl.kernel,
        mesh=_mesh(),
        out_type=jax.ShapeDtypeStruct((NCORE, N_PAD, 32), jnp.float32),
        scratch_types=[
            pltpu.VMEM((k_tile, CHUNK), jnp.int32),
            pltpu.VMEM((k_tile, CHUNK), jnp.int32),
            pltpu.VMEM((2 * NBUF, CHUNK, 32), jnp.float32),
            pltpu.VMEM_SHARED((N_PAD, 32), jnp.float32),
            pltpu.VMEM_SHARED((N_PAD, 32), jnp.float32),
        ] + [pltpu.SemaphoreType.DMA] * (4 * NBUF),
        compiler_params=pltpu.CompilerParams(use_tc_tiling_on_sc=False),
    )
    def prop2_k(u_hbm, ei_hbm, zeros_hbm, out_hbm,
                src_v, dst_v, rows_v, acc_sh, u_sh, *sems):
        gsem = sems[:2 * NBUF]
        ssem = sems[2 * NBUF:]
        c = lax.axis_index("c")
        s = lax.axis_index("s")
        wid = c * NSUB + s
        r0 = pl.multiple_of(s * ROWS_PER_TILE, 8)
        pltpu.sync_copy(zeros_hbm.at[pl.ds(r0, ROWS_PER_TILE)],
                        acc_sh.at[pl.ds(r0, ROWS_PER_TILE)])
        pltpu.sync_copy(ei_hbm.at[0, pl.ds(wid * k_tile, k_tile)], src_v)
        pltpu.sync_copy(ei_hbm.at[1, pl.ds(wid * k_tile, k_tile)], dst_v)
        _stage_rows(u_hbm, u_sh, s)
        plsc.subcore_barrier()
        _gather_scatter_loop(u_sh, acc_sh, src_v, dst_v, rows_v, gsem, ssem,
                             k_tile)
        plsc.subcore_barrier()
        pltpu.sync_copy(acc_sh.at[pl.ds(r0, ROWS_PER_TILE)],
                        out_hbm.at[c, pl.ds(r0, ROWS_PER_TILE)])

    return prop2_k(u, ei_p, zeros)


def _tc1_call(bins, x, W1):
    def body(bins_ref, x_ref, w_ref, ua_ref, ub_ref, dis_ref):
        deg = bins_ref[0][:, 0:1] + bins_ref[1][:, 0:1] + 1.0
        dis = lax.rsqrt(deg)
        h = jnp.dot(x_ref[...], w_ref[...], preferred_element_type=jnp.float32)
        u = h * dis
        ua_ref[...] = u[:, :32]
        ub_ref[...] = u[:, 32:]
        dis_ref[...] = dis

    return pl.pallas_call(
        body,
        grid=(N_NODES // BLK,),
        in_specs=[
            pl.BlockSpec((NCORE, BLK, 16), lambda i: (0, i, 0)),
            pl.BlockSpec((BLK, 128), lambda i: (i, 0)),
            pl.BlockSpec((128, 64), lambda i: (0, 0)),
        ],
        out_specs=[
            pl.BlockSpec((BLK, 32), lambda i: (i, 0)),
            pl.BlockSpec((BLK, 32), lambda i: (i, 0)),
            pl.BlockSpec((BLK, 1), lambda i: (i, 0)),
        ],
        out_shape=[
            jax.ShapeDtypeStruct((N_NODES, 32), jnp.float32),
            jax.ShapeDtypeStruct((N_NODES, 32), jnp.float32),
            jax.ShapeDtypeStruct((N_NODES, 1), jnp.float32),
        ],
    )(bins, x, W1)


def _tc2_call(p1, u1a, u1b, dis, b1, W2):
    def body(p_ref, ua_ref, ub_ref, dis_ref, b1_ref, w2_ref, u2_ref):
        dis = dis_ref[...]
        sa = p_ref[0] + ua_ref[...]
        sb = p_ref[1] + ub_ref[...]
        oa = jnp.maximum(sa * dis + b1_ref[:, :32], 0.0)
        ob = jnp.maximum(sb * dis + b1_ref[:, 32:], 0.0)
        u2 = (jnp.dot(oa, w2_ref[0], preferred_element_type=jnp.float32)
              + jnp.dot(ob, w2_ref[1], preferred_element_type=jnp.float32))
        u2_ref[...] = u2 * dis

    return pl.pallas_call(
        body,
        grid=(N_NODES // BLK,),
        in_specs=[
            pl.BlockSpec((NCORE, BLK, 32), lambda i: (0, i, 0)),
            pl.BlockSpec((BLK, 32), lambda i: (i, 0)),
            pl.BlockSpec((BLK, 32), lambda i: (i, 0)),
            pl.BlockSpec((BLK, 1), lambda i: (i, 0)),
            pl.BlockSpec((1, 64), lambda i: (0, 0)),
            pl.BlockSpec((2, 32, 32), lambda i: (0, 0, 0)),
        ],
        out_specs=pl.BlockSpec((BLK, 32), lambda i: (i, 0)),
        out_shape=jax.ShapeDtypeStruct((N_NODES, 32), jnp.float32),
    )(p1, u1a, u1b, dis, b1, W2.reshape(2, 32, 32))


def _tc3_call(p, u2, dis, b2, batch_r):
    def body(p_ref, u2_ref, dis_ref, b2_ref, bt_ref, out_ref):
        h2 = (p_ref[0] + p_ref[1] + u2_ref[...]) * dis_ref[...] + b2_ref[...]
        bt = bt_ref[0]
        oh = (lax.broadcasted_iota(jnp.int32, (NUM_GRAPHS, BLK), 0)
              == bt).astype(jnp.float32)
        acc = jnp.dot(oh, h2, preferred_element_type=jnp.float32)
        i = pl.program_id(0)

        @pl.when(i == 0)
        def _():
            out_ref[...] = acc

        @pl.when(i != 0)
        def _():
            out_ref[...] += acc

    return pl.pallas_call(
        body,
        grid=(N_NODES // BLK,),
        in_specs=[
            pl.BlockSpec((NCORE, BLK, 32), lambda i: (0, i, 0)),
            pl.BlockSpec((BLK, 32), lambda i: (i, 0)),
            pl.BlockSpec((BLK, 1), lambda i: (i, 0)),
            pl.BlockSpec((1, 32), lambda i: (0, 0)),
            pl.BlockSpec((1, 1, BLK), lambda i: (i, 0, 0)),
        ],
        out_specs=pl.BlockSpec((NUM_GRAPHS, 32), lambda i: (0, 0)),
        out_shape=jax.ShapeDtypeStruct((NUM_GRAPHS, 32), jnp.float32),
    )(p, u2, dis, b2, batch_r)


def kernel(x, edge_index, batch, W1, b1, W2, b2):
    pad = CAP - NUM_EDGES
    pad_src = jnp.zeros((pad,), jnp.int32)
    pad_dst = N_NODES + (jnp.arange(pad, dtype=jnp.int32) % (N_PAD - N_NODES))
    ei_p = jnp.concatenate(
        [edge_index.astype(jnp.int32), jnp.stack([pad_src, pad_dst])],
        axis=1).reshape(2, N_CHUNKS, CHUNK)
    ones_rows = jnp.ones((CHUNK, 16), jnp.float32)
    z16 = jnp.zeros((N_PAD, 16), jnp.float32)
    z32 = jnp.zeros((N_PAD, 32), jnp.float32)

    bins = _deg_call(ei_p, ones_rows, z16)
    u1a, u1b, dis = _tc1_call(bins, x, W1)
    p1 = _prop1_call(u1a, u1b, ei_p, z32)
    u2 = _tc2_call(p1, u1a, u1b, dis, b1.reshape(1, 64), W2)
    p2 = _prop2_call(u2, ei_p, z32)
    out = _tc3_call(p2, u2, dis, b2.reshape(1, 32),
                    batch.astype(jnp.int32).reshape(N_NODES // BLK, 1, BLK))
    return out


# speedup vs baseline: 2.2645x; 1.0030x over previous
import functools

import jax
import jax.numpy as jnp
from jax import lax
from jax.experimental import pallas as pl
from jax.experimental.pallas import tpu as pltpu
from jax.experimental.pallas import tpu_sc as plsc

N_NODES = 10000
NUM_EDGES = 320000
NUM_GRAPHS = 64
NCORE = 2
NSUB = 16
NW = NCORE * NSUB
CHUNK = 128
N_CHUNKS = 2560
NBUF = 4
CAP = N_CHUNKS * CHUNK
N_PAD = 10112
ROWS_PER_TILE = N_PAD // NSUB
BLK = 1000


def _mesh():
    return plsc.VectorSubcoreMesh(core_axis_name="c", subcore_axis_name="s")


def _stage_rows(u_hbm, u_sh, s):
    r0 = pl.multiple_of(s * ROWS_PER_TILE, 8)
    nfull = N_NODES // ROWS_PER_TILE
    rem = N_NODES - nfull * ROWS_PER_TILE

    @pl.when(s < nfull)
    def _():
        pltpu.sync_copy(u_hbm.at[pl.ds(r0, ROWS_PER_TILE)],
                        u_sh.at[pl.ds(r0, ROWS_PER_TILE)])

    @pl.when(s == nfull)
    def _():
        rr = pl.multiple_of(nfull * ROWS_PER_TILE, 8)
        pltpu.sync_copy(u_hbm.at[pl.ds(rr, rem)], u_sh.at[pl.ds(rr, rem)])


def _gather_scatter_loop(u_src, acc_sh, src_v, dst_v, rows_v, gsem, ssem,
                         k_tile):
    nstep = k_tile // (2 * NBUF)

    def fire_gather(slot, j):
        pltpu.async_copy(u_src.at[src_v.at[j]], rows_v.at[slot], gsem[slot])

    def wait_gather(slot, j):
        pltpu.make_async_copy(u_src.at[src_v.at[j]], rows_v.at[slot],
                              gsem[slot]).wait()

    for b in range(2 * NBUF):
        fire_gather(b, b)

    def body(t, carry):
        j0 = 2 * NBUF * t
        for half in range(2):
            off = half * NBUF
            for b in range(NBUF):
                wait_gather(off + b, j0 + off + b)
            scps = [
                pltpu.async_copy(rows_v.at[off + b],
                                 acc_sh.at[dst_v.at[j0 + off + b]],
                                 ssem[off + b], add=True)
                for b in range(NBUF)
            ]
            for d_ in scps:
                d_.wait()

            @pl.when(t < nstep - 1)
            def _():
                for b in range(NBUF):
                    fire_gather(off + b, j0 + 2 * NBUF + off + b)

        return carry

    lax.fori_loop(0, nstep, body, 0)


def _deg_call(ei_p, ones_rows, zeros16):
    k_tile = N_CHUNKS // NW

    @functools.partial(
        pl.kernel,
        mesh=_mesh(),
        out_type=jax.ShapeDtypeStruct((NCORE, N_PAD, 16), jnp.float32),
        scratch_types=[
            pltpu.VMEM((k_tile, CHUNK), jnp.int32),
            pltpu.VMEM((CHUNK, 16), jnp.float32),
            pltpu.VMEM_SHARED((N_PAD, 16), jnp.float32),
            pltpu.SemaphoreType.DMA,
        ],
        compiler_params=pltpu.CompilerParams(use_tc_tiling_on_sc=False),
    )
    def deg_k(ei_hbm, ones_hbm, zeros_hbm, out_hbm, idx_v, ones_v, bins_sh,
              sem):
        c = lax.axis_index("c")
        s = lax.axis_index("s")
        wid = c * NSUB + s
        r0 = pl.multiple_of(s * ROWS_PER_TILE, 8)
        pltpu.sync_copy(zeros_hbm.at[pl.ds(r0, ROWS_PER_TILE)],
                        bins_sh.at[pl.ds(r0, ROWS_PER_TILE)])
        pltpu.sync_copy(ones_hbm, ones_v)
        pltpu.sync_copy(ei_hbm.at[1, pl.ds(wid * k_tile, k_tile)], idx_v)
        plsc.subcore_barrier()

        def body(t, carry):
            for b in range(8):
                pltpu.async_copy(ones_v, bins_sh.at[idx_v.at[t * 8 + b]],
                                 sem, add=True)

            @pl.when(t > 0)
            def _():
                for b in range(8):
                    pltpu.make_async_copy(
                        ones_v, bins_sh.at[idx_v.at[b]], sem).wait()

            return carry

        lax.fori_loop(0, k_tile // 8, body, 0)
        for b in range(8):
            pltpu.make_async_copy(ones_v, bins_sh.at[idx_v.at[b]], sem).wait()
        plsc.subcore_barrier()
        pltpu.sync_copy(bins_sh.at[pl.ds(r0, ROWS_PER_TILE)],
                        out_hbm.at[c, pl.ds(r0, ROWS_PER_TILE)])

    return deg_k(ei_p, ones_rows, zeros16)


def _prop1_call(ua, ub, ei_p):
    k_tile = N_CHUNKS // NSUB

    @functools.partial(
        pl.kernel,
        mesh=_mesh(),
        out_type=jax.ShapeDtypeStruct((NCORE, N_PAD, 32), jnp.float32),
        scratch_types=[
            pltpu.VMEM((k_tile, CHUNK), jnp.int32),
            pltpu.VMEM((k_tile, CHUNK), jnp.int32),
            pltpu.VMEM((2 * NBUF, CHUNK, 32), jnp.float32),
            pltpu.VMEM_SHARED((N_PAD, 32), jnp.float32),
            pltpu.VMEM_SHARED((N_PAD, 32), jnp.float32),
        ] + [pltpu.SemaphoreType.DMA] * (4 * NBUF),
        compiler_params=pltpu.CompilerParams(use_tc_tiling_on_sc=False),
    )
    def prop1_k(ua_hbm, ub_hbm, ei_hbm, out_hbm,
                src_v, dst_v, rows_v, acc_sh, u_sh, *sems):
        gsem = sems[:2 * NBUF]
        ssem = sems[2 * NBUF:]
        c = lax.axis_index("c")
        s = lax.axis_index("s")
        r0 = pl.multiple_of(s * ROWS_PER_TILE, 8)
        pltpu.sync_copy(ei_hbm.at[0, pl.ds(s * k_tile, k_tile)], src_v)
        pltpu.sync_copy(ei_hbm.at[1, pl.ds(s * k_tile, k_tile)], dst_v)

        @pl.when(c == 0)
        def _():
            _stage_rows(ua_hbm, u_sh, s)
            _stage_rows(ua_hbm, acc_sh, s)

        @pl.when(c == 1)
        def _():
            _stage_rows(ub_hbm, u_sh, s)
            _stage_rows(ub_hbm, acc_sh, s)

        plsc.subcore_barrier()
        _gather_scatter_loop(u_sh, acc_sh, src_v, dst_v, rows_v, gsem, ssem,
                             k_tile)
        plsc.subcore_barrier()
        pltpu.sync_copy(acc_sh.at[pl.ds(r0, ROWS_PER_TILE)],
                        out_hbm.at[c, pl.ds(r0, ROWS_PER_TILE)])

    return prop1_k(ua, ub, ei_p)


def _prop2_call(u, ei_p, zeros):
    k_tile = N_CHUNKS // NW

    @functools.partial(
        pl.kernel,
        mesh=_mesh(),
        out_type=jax.ShapeDtypeStruct((NCORE, N_PAD, 32), jnp.float32),
        scratch_types=[
            pltpu.VMEM((k_tile, CHUNK), jnp.int32),
            pltpu.VMEM((k_tile, CHUNK), jnp.int32),
            pltpu.VMEM((2 * NBUF, CHUNK, 32), jnp.float32),
            pltpu.VMEM_SHARED((N_PAD, 32), jnp.float32),
            pltpu.VMEM_SHARED((N_PAD, 32), jnp.float32),
        ] + [pltpu.SemaphoreType.DMA] * (4 * NBUF),
        compiler_params=pltpu.CompilerParams(use_tc_tiling_on_sc=False),
    )
    def prop2_k(u_hbm, ei_hbm, zeros_hbm, out_hbm,
                src_v, dst_v, rows_v, acc_sh, u_sh, *sems):
        gsem = sems[:2 * NBUF]
        ssem = sems[2 * NBUF:]
        c = lax.axis_index("c")
        s = lax.axis_index("s")
        wid = c * NSUB + s
        r0 = pl.multiple_of(s * ROWS_PER_TILE, 8)
        pltpu.sync_copy(ei_hbm.at[0, pl.ds(wid * k_tile, k_tile)], src_v)
        pltpu.sync_copy(ei_hbm.at[1, pl.ds(wid * k_tile, k_tile)], dst_v)
        _stage_rows(u_hbm, u_sh, s)

        @pl.when(c == 0)
        def _():
            _stage_rows(u_hbm, acc_sh, s)

        @pl.when(c == 1)
        def _():
            pltpu.sync_copy(zeros_hbm.at[pl.ds(r0, ROWS_PER_TILE)],
                            acc_sh.at[pl.ds(r0, ROWS_PER_TILE)])

        plsc.subcore_barrier()
        _gather_scatter_loop(u_sh, acc_sh, src_v, dst_v, rows_v, gsem, ssem,
                             k_tile)
        plsc.subcore_barrier()
        pltpu.sync_copy(acc_sh.at[pl.ds(r0, ROWS_PER_TILE)],
                        out_hbm.at[c, pl.ds(r0, ROWS_PER_TILE)])

    return prop2_k(u, ei_p, zeros)


def _tc1m_call(x, W1):
    def body(x_ref, w_ref, h_ref):
        h_ref[...] = jnp.dot(x_ref[...], w_ref[...],
                             preferred_element_type=jnp.float32)

    return pl.pallas_call(
        body,
        grid=(N_NODES // BLK,),
        in_specs=[
            pl.BlockSpec((BLK, 128), lambda i: (i, 0)),
            pl.BlockSpec((128, 64), lambda i: (0, 0)),
        ],
        out_specs=pl.BlockSpec((BLK, 64), lambda i: (i, 0)),
        out_shape=jax.ShapeDtypeStruct((N_NODES, 64), jnp.float32),
    )(x, W1)


def _tc1s_call(bins, h):
    def body(bins_ref, h_ref, ua_ref, ub_ref, dis_ref):
        deg = bins_ref[0][:, 0:1] + bins_ref[1][:, 0:1] + 1.0
        dis = lax.rsqrt(deg)
        u = h_ref[...] * dis
        ua_ref[...] = u[:, :32]
        ub_ref[...] = u[:, 32:]
        dis_ref[...] = dis

    return pl.pallas_call(
        body,
        grid=(N_NODES // BLK,),
        in_specs=[
            pl.BlockSpec((NCORE, BLK, 16), lambda i: (0, i, 0)),
            pl.BlockSpec((BLK, 64), lambda i: (i, 0)),
        ],
        out_specs=[
            pl.BlockSpec((BLK, 32), lambda i: (i, 0)),
            pl.BlockSpec((BLK, 32), lambda i: (i, 0)),
            pl.BlockSpec((BLK, 1), lambda i: (i, 0)),
        ],
        out_shape=[
            jax.ShapeDtypeStruct((N_NODES, 32), jnp.float32),
            jax.ShapeDtypeStruct((N_NODES, 32), jnp.float32),
            jax.ShapeDtypeStruct((N_NODES, 1), jnp.float32),
        ],
    )(bins, h)


def _tc2_call(p1, dis, b1, W2):
    def body(p_ref, dis_ref, b1_ref, w2_ref, u2_ref):
        dis = dis_ref[...]
        oa = jnp.maximum(p_ref[0] * dis + b1_ref[:, :32], 0.0)
        ob = jnp.maximum(p_ref[1] * dis + b1_ref[:, 32:], 0.0)
        u2 = (jnp.dot(oa, w2_ref[0], preferred_element_type=jnp.float32)
              + jnp.dot(ob, w2_ref[1], preferred_element_type=jnp.float32))
        u2_ref[...] = u2 * dis

    return pl.pallas_call(
        body,
        grid=(N_NODES // BLK,),
        in_specs=[
            pl.BlockSpec((NCORE, BLK, 32), lambda i: (0, i, 0)),
            pl.BlockSpec((BLK, 1), lambda i: (i, 0)),
            pl.BlockSpec((1, 64), lambda i: (0, 0)),
            pl.BlockSpec((2, 32, 32), lambda i: (0, 0, 0)),
        ],
        out_specs=pl.BlockSpec((BLK, 32), lambda i: (i, 0)),
        out_shape=jax.ShapeDtypeStruct((N_NODES, 32), jnp.float32),
    )(p1, dis, b1, W2.reshape(2, 32, 32))


def _tc3_call(p, dis, b2, batch_r):
    def body(p_ref, dis_ref, b2_ref, bt_ref, out_ref):
        h2 = (p_ref[0] + p_ref[1]) * dis_ref[...] + b2_ref[...]
        bt = bt_ref[0]
        oh = (lax.broadcasted_iota(jnp.int32, (NUM_GRAPHS, BLK), 0)
              == bt).astype(jnp.float32)
        acc = jnp.dot(oh, h2, preferred_element_type=jnp.float32)
        i = pl.program_id(0)

        @pl.when(i == 0)
        def _():
            out_ref[...] = acc

        @pl.when(i != 0)
        def _():
            out_ref[...] += acc

    return pl.pallas_call(
        body,
        grid=(N_NODES // BLK,),
        in_specs=[
            pl.BlockSpec((NCORE, BLK, 32), lambda i: (0, i, 0)),
            pl.BlockSpec((BLK, 1), lambda i: (i, 0)),
            pl.BlockSpec((1, 32), lambda i: (0, 0)),
            pl.BlockSpec((1, 1, BLK), lambda i: (i, 0, 0)),
        ],
        out_specs=pl.BlockSpec((NUM_GRAPHS, 32), lambda i: (0, 0)),
        out_shape=jax.ShapeDtypeStruct((NUM_GRAPHS, 32), jnp.float32),
    )(p, dis, b2, batch_r)


def kernel(x, edge_index, batch, W1, b1, W2, b2):
    pad = CAP - NUM_EDGES
    pad_src = jnp.zeros((pad,), jnp.int32)
    pad_dst = N_NODES + (jnp.arange(pad, dtype=jnp.int32) % (N_PAD - N_NODES))
    ei_p = jnp.concatenate(
        [edge_index.astype(jnp.int32), jnp.stack([pad_src, pad_dst])],
        axis=1).reshape(2, N_CHUNKS, CHUNK)
    ones_rows = jnp.ones((CHUNK, 16), jnp.float32)
    z16 = jnp.zeros((N_PAD, 16), jnp.float32)
    z32 = jnp.zeros((N_PAD, 32), jnp.float32)

    h = _tc1m_call(x, W1)
    bins = _deg_call(ei_p, ones_rows, z16)
    u1a, u1b, dis = _tc1s_call(bins, h)
    p1 = _prop1_call(u1a, u1b, ei_p)
    u2 = _tc2_call(p1, dis, b1.reshape(1, 64), W2)
    p2 = _prop2_call(u2, ei_p, z32)
    out = _tc3_call(p2, dis, b2.reshape(1, 32),
                    batch.astype(jnp.int32).reshape(N_NODES // BLK, 1, BLK))
    return out


# speedup vs baseline: 2.5242x; 1.1147x over previous
import functools

import jax
import jax.numpy as jnp
from jax import lax
from jax.experimental import pallas as pl
from jax.experimental.pallas import tpu as pltpu
from jax.experimental.pallas import tpu_sc as plsc

N_NODES = 10000
NUM_EDGES = 320000
NUM_GRAPHS = 64
NCORE = 2
NSUB = 16
NW = NCORE * NSUB
CHUNK = 128
N_CHUNKS = 2560
NBUF = 4
CAP = N_CHUNKS * CHUNK
N_PAD = 10112
ROWS_PER_TILE = N_PAD // NSUB
BLK = 2000


def _mesh():
    return plsc.VectorSubcoreMesh(core_axis_name="c", subcore_axis_name="s")


def _stage_rows(u_hbm, u_sh, s):
    r0 = pl.multiple_of(s * ROWS_PER_TILE, 8)
    nfull = N_NODES // ROWS_PER_TILE
    rem = N_NODES - nfull * ROWS_PER_TILE

    @pl.when(s < nfull)
    def _():
        pltpu.sync_copy(u_hbm.at[pl.ds(r0, ROWS_PER_TILE)],
                        u_sh.at[pl.ds(r0, ROWS_PER_TILE)])

    @pl.when(s == nfull)
    def _():
        rr = pl.multiple_of(nfull * ROWS_PER_TILE, 8)
        pltpu.sync_copy(u_hbm.at[pl.ds(rr, rem)], u_sh.at[pl.ds(rr, rem)])


def _gather_scatter_loop(u_src, acc_sh, src_v, dst_v, rows_v, gsem, ssem,
                         k_tile):
    nstep = k_tile // (2 * NBUF)

    def fire_gather(slot, j):
        pltpu.async_copy(u_src.at[src_v.at[j]], rows_v.at[slot], gsem[slot])

    def wait_gather(slot, j):
        pltpu.make_async_copy(u_src.at[src_v.at[j]], rows_v.at[slot],
                              gsem[slot]).wait()

    for b in range(2 * NBUF):
        fire_gather(b, b)

    def body(t, carry):
        j0 = 2 * NBUF * t
        for half in range(2):
            off = half * NBUF
            for b in range(NBUF):
                wait_gather(off + b, j0 + off + b)
            scps = [
                pltpu.async_copy(rows_v.at[off + b],
                                 acc_sh.at[dst_v.at[j0 + off + b]],
                                 ssem[off + b], add=True)
                for b in range(NBUF)
            ]
            for d_ in scps:
                d_.wait()

            @pl.when(t < nstep - 1)
            def _():
                for b in range(NBUF):
                    fire_gather(off + b, j0 + 2 * NBUF + off + b)

        return carry

    lax.fori_loop(0, nstep, body, 0)


def _deg_call(ei_p, ones_rows, zeros16):
    k_tile = N_CHUNKS // NW

    @functools.partial(
        pl.kernel,
        mesh=_mesh(),
        out_type=jax.ShapeDtypeStruct((NCORE, N_PAD, 16), jnp.float32),
        scratch_types=[
            pltpu.VMEM((k_tile, CHUNK), jnp.int32),
            pltpu.VMEM((CHUNK, 16), jnp.float32),
            pltpu.VMEM_SHARED((N_PAD, 16), jnp.float32),
            pltpu.SemaphoreType.DMA,
        ],
        compiler_params=pltpu.CompilerParams(use_tc_tiling_on_sc=False),
    )
    def deg_k(ei_hbm, ones_hbm, zeros_hbm, out_hbm, idx_v, ones_v, bins_sh,
              sem):
        c = lax.axis_index("c")
        s = lax.axis_index("s")
        wid = c * NSUB + s
        r0 = pl.multiple_of(s * ROWS_PER_TILE, 8)
        pltpu.sync_copy(zeros_hbm.at[pl.ds(r0, ROWS_PER_TILE)],
                        bins_sh.at[pl.ds(r0, ROWS_PER_TILE)])
        pltpu.sync_copy(ones_hbm, ones_v)
        pltpu.sync_copy(ei_hbm.at[1, pl.ds(wid * k_tile, k_tile)], idx_v)
        plsc.subcore_barrier()

        def body(t, carry):
            for b in range(8):
                pltpu.async_copy(ones_v, bins_sh.at[idx_v.at[t * 8 + b]],
                                 sem, add=True)

            @pl.when(t > 0)
            def _():
                for b in range(8):
                    pltpu.make_async_copy(
                        ones_v, bins_sh.at[idx_v.at[b]], sem).wait()

            return carry

        lax.fori_loop(0, k_tile // 8, body, 0)
        for b in range(8):
            pltpu.make_async_copy(ones_v, bins_sh.at[idx_v.at[b]], sem).wait()
        plsc.subcore_barrier()
        pltpu.sync_copy(bins_sh.at[pl.ds(r0, ROWS_PER_TILE)],
                        out_hbm.at[c, pl.ds(r0, ROWS_PER_TILE)])

    return deg_k(ei_p, ones_rows, zeros16)


def _prop1_call(ua, ub, ei_p):
    k_tile = N_CHUNKS // NSUB

    @functools.partial(
        pl.kernel,
        mesh=_mesh(),
        out_type=jax.ShapeDtypeStruct((NCORE, N_PAD, 32), jnp.float32),
        scratch_types=[
            pltpu.VMEM((k_tile, CHUNK), jnp.int32),
            pltpu.VMEM((k_tile, CHUNK), jnp.int32),
            pltpu.VMEM((2 * NBUF, CHUNK, 32), jnp.float32),
            pltpu.VMEM_SHARED((N_PAD, 32), jnp.float32),
            pltpu.VMEM_SHARED((N_PAD, 32), jnp.float32),
        ] + [pltpu.SemaphoreType.DMA] * (4 * NBUF),
        compiler_params=pltpu.CompilerParams(use_tc_tiling_on_sc=False),
    )
    def prop1_k(ua_hbm, ub_hbm, ei_hbm, out_hbm,
                src_v, dst_v, rows_v, acc_sh, u_sh, *sems):
        gsem = sems[:2 * NBUF]
        ssem = sems[2 * NBUF:]
        c = lax.axis_index("c")
        s = lax.axis_index("s")
        r0 = pl.multiple_of(s * ROWS_PER_TILE, 8)
        pltpu.sync_copy(ei_hbm.at[0, pl.ds(s * k_tile, k_tile)], src_v)
        pltpu.sync_copy(ei_hbm.at[1, pl.ds(s * k_tile, k_tile)], dst_v)

        @pl.when(c == 0)
        def _():
            _stage_rows(ua_hbm, u_sh, s)
            _stage_rows(ua_hbm, acc_sh, s)

        @pl.when(c == 1)
        def _():
            _stage_rows(ub_hbm, u_sh, s)
            _stage_rows(ub_hbm, acc_sh, s)

        plsc.subcore_barrier()
        _gather_scatter_loop(u_sh, acc_sh, src_v, dst_v, rows_v, gsem, ssem,
                             k_tile)
        plsc.subcore_barrier()
        pltpu.sync_copy(acc_sh.at[pl.ds(r0, ROWS_PER_TILE)],
                        out_hbm.at[c, pl.ds(r0, ROWS_PER_TILE)])

    return prop1_k(ua, ub, ei_p)


def _prop2_call(u, ei_p, zeros):
    k_tile = N_CHUNKS // NW

    @functools.partial(
        pl.kernel,
        mesh=_mesh(),
        out_type=jax.ShapeDtypeStruct((NCORE, N_PAD, 32), jnp.float32),
        scratch_types=[
            pltpu.VMEM((k_tile, CHUNK), jnp.int32),
            pltpu.VMEM((k_tile, CHUNK), jnp.int32),
            pltpu.VMEM((2 * NBUF, CHUNK, 32), jnp.float32),
            pltpu.VMEM_SHARED((N_PAD, 32), jnp.float32),
            pltpu.VMEM_SHARED((N_PAD, 32), jnp.float32),
        ] + [pltpu.SemaphoreType.DMA] * (4 * NBUF),
        compiler_params=pltpu.CompilerParams(use_tc_tiling_on_sc=False),
    )
    def prop2_k(u_hbm, ei_hbm, zeros_hbm, out_hbm,
                src_v, dst_v, rows_v, acc_sh, u_sh, *sems):
        gsem = sems[:2 * NBUF]
        ssem = sems[2 * NBUF:]
        c = lax.axis_index("c")
        s = lax.axis_index("s")
        wid = c * NSUB + s
        r0 = pl.multiple_of(s * ROWS_PER_TILE, 8)
        pltpu.sync_copy(ei_hbm.at[0, pl.ds(wid * k_tile, k_tile)], src_v)
        pltpu.sync_copy(ei_hbm.at[1, pl.ds(wid * k_tile, k_tile)], dst_v)
        _stage_rows(u_hbm, u_sh, s)

        @pl.when(c == 0)
        def _():
            _stage_rows(u_hbm, acc_sh, s)

        @pl.when(c == 1)
        def _():
            pltpu.sync_copy(zeros_hbm.at[pl.ds(r0, ROWS_PER_TILE)],
                            acc_sh.at[pl.ds(r0, ROWS_PER_TILE)])

        plsc.subcore_barrier()
        _gather_scatter_loop(u_sh, acc_sh, src_v, dst_v, rows_v, gsem, ssem,
                             k_tile)
        plsc.subcore_barrier()
        pltpu.sync_copy(acc_sh.at[pl.ds(r0, ROWS_PER_TILE)],
                        out_hbm.at[c, pl.ds(r0, ROWS_PER_TILE)])

    return prop2_k(u, ei_p, zeros)


def _tc1m_call(x, W1):
    def body(x_ref, w_ref, h_ref):
        h_ref[...] = jnp.dot(x_ref[...], w_ref[...],
                             preferred_element_type=jnp.float32)

    return pl.pallas_call(
        body,
        grid=(N_NODES // BLK,),
        in_specs=[
            pl.BlockSpec((BLK, 128), lambda i: (i, 0)),
            pl.BlockSpec((128, 64), lambda i: (0, 0)),
        ],
        out_specs=pl.BlockSpec((BLK, 64), lambda i: (i, 0)),
        out_shape=jax.ShapeDtypeStruct((N_NODES, 64), jnp.float32),
    )(x, W1)


def _tc1s_call(bins, h):
    def body(bins_ref, h_ref, ua_ref, ub_ref, dis_ref):
        deg = bins_ref[0][:, 0:1] + bins_ref[1][:, 0:1] + 1.0
        dis = lax.rsqrt(deg)
        u = h_ref[...] * dis
        ua_ref[...] = u[:, :32]
        ub_ref[...] = u[:, 32:]
        dis_ref[...] = dis

    return pl.pallas_call(
        body,
        grid=(N_NODES // BLK,),
        in_specs=[
            pl.BlockSpec((NCORE, BLK, 16), lambda i: (0, i, 0)),
            pl.BlockSpec((BLK, 64), lambda i: (i, 0)),
        ],
        out_specs=[
            pl.BlockSpec((BLK, 32), lambda i: (i, 0)),
            pl.BlockSpec((BLK, 32), lambda i: (i, 0)),
            pl.BlockSpec((BLK, 1), lambda i: (i, 0)),
        ],
        out_shape=[
            jax.ShapeDtypeStruct((N_NODES, 32), jnp.float32),
            jax.ShapeDtypeStruct((N_NODES, 32), jnp.float32),
            jax.ShapeDtypeStruct((N_NODES, 1), jnp.float32),
        ],
    )(bins, h)


def _tc2_call(p1, dis, b1, W2):
    def body(p_ref, dis_ref, b1_ref, w2_ref, u2_ref):
        dis = dis_ref[...]
        oa = jnp.maximum(p_ref[0] * dis + b1_ref[:, :32], 0.0)
        ob = jnp.maximum(p_ref[1] * dis + b1_ref[:, 32:], 0.0)
        u2 = (jnp.dot(oa, w2_ref[0], preferred_element_type=jnp.float32)
              + jnp.dot(ob, w2_ref[1], preferred_element_type=jnp.float32))
        u2_ref[...] = u2 * dis

    return pl.pallas_call(
        body,
        grid=(N_NODES // BLK,),
        in_specs=[
            pl.BlockSpec((NCORE, BLK, 32), lambda i: (0, i, 0)),
            pl.BlockSpec((BLK, 1), lambda i: (i, 0)),
            pl.BlockSpec((1, 64), lambda i: (0, 0)),
            pl.BlockSpec((2, 32, 32), lambda i: (0, 0, 0)),
        ],
        out_specs=pl.BlockSpec((BLK, 32), lambda i: (i, 0)),
        out_shape=jax.ShapeDtypeStruct((N_NODES, 32), jnp.float32),
    )(p1, dis, b1, W2.reshape(2, 32, 32))


def _tc3_call(p, dis, b2, batch_r):
    def body(p_ref, dis_ref, b2_ref, bt_ref, out_ref):
        h2 = (p_ref[0] + p_ref[1]) * dis_ref[...] + b2_ref[...]
        bt = bt_ref[0]
        oh = (lax.broadcasted_iota(jnp.int32, (NUM_GRAPHS, BLK), 0)
              == bt).astype(jnp.float32)
        acc = jnp.dot(oh, h2, preferred_element_type=jnp.float32)
        i = pl.program_id(0)

        @pl.when(i == 0)
        def _():
            out_ref[...] = acc

        @pl.when(i != 0)
        def _():
            out_ref[...] += acc

    return pl.pallas_call(
        body,
        grid=(N_NODES // BLK,),
        in_specs=[
            pl.BlockSpec((NCORE, BLK, 32), lambda i: (0, i, 0)),
            pl.BlockSpec((BLK, 1), lambda i: (i, 0)),
            pl.BlockSpec((1, 32), lambda i: (0, 0)),
            pl.BlockSpec((1, 1, BLK), lambda i: (i, 0, 0)),
        ],
        out_specs=pl.BlockSpec((NUM_GRAPHS, 32), lambda i: (0, 0)),
        out_shape=jax.ShapeDtypeStruct((NUM_GRAPHS, 32), jnp.float32),
    )(p, dis, b2, batch_r)


def kernel(x, edge_index, batch, W1, b1, W2, b2):
    pad = CAP - NUM_EDGES
    pad_src = jnp.arange(pad, dtype=jnp.int32) % N_NODES
    pad_dst = N_NODES + (jnp.arange(pad, dtype=jnp.int32) % (N_PAD - N_NODES))
    ei_p = jnp.concatenate(
        [edge_index.astype(jnp.int32), jnp.stack([pad_src, pad_dst])],
        axis=1).reshape(2, N_CHUNKS, CHUNK)
    ones_rows = jnp.ones((CHUNK, 16), jnp.float32)
    z16 = jnp.zeros((N_PAD, 16), jnp.float32)
    z32 = jnp.zeros((N_PAD, 32), jnp.float32)

    h = _tc1m_call(x, W1)
    bins = _deg_call(ei_p, ones_rows, z16)
    u1a, u1b, dis = _tc1s_call(bins, h)
    p1 = _prop1_call(u1a, u1b, ei_p)
    u2 = _tc2_call(p1, dis, b1.reshape(1, 64), W2)
    p2 = _prop2_call(u2, ei_p, z32)
    out = _tc3_call(p2, dis, b2.reshape(1, 32),
                    batch.astype(jnp.int32).reshape(N_NODES // BLK, 1, BLK))
    return out


# speedup vs baseline: 2.6395x; 1.0457x over previous
import functools

import jax
import jax.numpy as jnp
from jax import lax
from jax.experimental import pallas as pl
from jax.experimental.pallas import tpu as pltpu
from jax.experimental.pallas import tpu_sc as plsc

N_NODES = 10000
NUM_EDGES = 320000
NUM_GRAPHS = 64
NCORE = 2
NSUB = 16
NW = NCORE * NSUB
CHUNK = 128
N_CHUNKS = 2560
NBUF = 4
CAP = N_CHUNKS * CHUNK
N_PAD = 10112
ROWS_PER_TILE = N_PAD // NSUB
BLK = 2000


def _mesh():
    return plsc.VectorSubcoreMesh(core_axis_name="c", subcore_axis_name="s")


def _stage_rows(u_hbm, u_sh, s):
    r0 = pl.multiple_of(s * ROWS_PER_TILE, 8)
    nfull = N_NODES // ROWS_PER_TILE
    rem = N_NODES - nfull * ROWS_PER_TILE

    @pl.when(s < nfull)
    def _():
        pltpu.sync_copy(u_hbm.at[pl.ds(r0, ROWS_PER_TILE)],
                        u_sh.at[pl.ds(r0, ROWS_PER_TILE)])

    @pl.when(s == nfull)
    def _():
        rr = pl.multiple_of(nfull * ROWS_PER_TILE, 8)
        pltpu.sync_copy(u_hbm.at[pl.ds(rr, rem)], u_sh.at[pl.ds(rr, rem)])


def _gather_scatter_loop(u_src, acc_sh, src_v, dst_v, rows_v, gsem, ssem,
                         k_tile):
    nstep = k_tile // (2 * NBUF)

    def fire_gather(slot, j):
        pltpu.async_copy(u_src.at[src_v.at[j]], rows_v.at[slot], gsem[slot])

    def wait_gather(slot, j):
        pltpu.make_async_copy(u_src.at[src_v.at[j]], rows_v.at[slot],
                              gsem[slot]).wait()

    for b in range(2 * NBUF):
        fire_gather(b, b)

    def body(t, carry):
        j0 = 2 * NBUF * t
        for half in range(2):
            off = half * NBUF
            for b in range(NBUF):
                wait_gather(off + b, j0 + off + b)
            scps = [
                pltpu.async_copy(rows_v.at[off + b],
                                 acc_sh.at[dst_v.at[j0 + off + b]],
                                 ssem[off + b], add=True)
                for b in range(NBUF)
            ]
            for d_ in scps:
                d_.wait()

            @pl.when(t < nstep - 1)
            def _():
                for b in range(NBUF):
                    fire_gather(off + b, j0 + 2 * NBUF + off + b)

        return carry

    lax.fori_loop(0, nstep, body, 0)


def _deg_call(ei_p, ones_rows, zeros16):
    k_tile = N_CHUNKS // NW

    @functools.partial(
        pl.kernel,
        mesh=_mesh(),
        out_type=jax.ShapeDtypeStruct((NCORE, N_PAD, 16), jnp.float32),
        scratch_types=[
            pltpu.VMEM((k_tile, CHUNK), jnp.int32),
            pltpu.VMEM((CHUNK, 16), jnp.float32),
            pltpu.VMEM_SHARED((N_PAD, 16), jnp.float32),
            pltpu.SemaphoreType.DMA,
        ],
        compiler_params=pltpu.CompilerParams(use_tc_tiling_on_sc=False),
    )
    def deg_k(ei_hbm, ones_hbm, zeros_hbm, out_hbm, idx_v, ones_v, bins_sh,
              sem):
        c = lax.axis_index("c")
        s = lax.axis_index("s")
        wid = c * NSUB + s
        r0 = pl.multiple_of(s * ROWS_PER_TILE, 8)
        pltpu.sync_copy(zeros_hbm.at[pl.ds(r0, ROWS_PER_TILE)],
                        bins_sh.at[pl.ds(r0, ROWS_PER_TILE)])
        pltpu.sync_copy(ones_hbm, ones_v)
        pltpu.sync_copy(ei_hbm.at[1, pl.ds(wid * k_tile, k_tile)], idx_v)
        plsc.subcore_barrier()

        def body(t, carry):
            for b in range(8):
                pltpu.async_copy(ones_v, bins_sh.at[idx_v.at[t * 8 + b]],
                                 sem, add=True)

            @pl.when(t > 0)
            def _():
                for b in range(8):
                    pltpu.make_async_copy(
                        ones_v, bins_sh.at[idx_v.at[b]], sem).wait()

            return carry

        lax.fori_loop(0, k_tile // 8, body, 0)
        for b in range(8):
            pltpu.make_async_copy(ones_v, bins_sh.at[idx_v.at[b]], sem).wait()
        plsc.subcore_barrier()
        pltpu.sync_copy(bins_sh.at[pl.ds(r0, ROWS_PER_TILE)],
                        out_hbm.at[c, pl.ds(r0, ROWS_PER_TILE)])

    return deg_k(ei_p, ones_rows, zeros16)


def _prop1_call(ua, ub, ei_p):
    k_tile = N_CHUNKS // NSUB

    @functools.partial(
        pl.kernel,
        mesh=_mesh(),
        out_type=jax.ShapeDtypeStruct((NCORE, N_PAD, 32), jnp.float32),
        scratch_types=[
            pltpu.VMEM((k_tile, CHUNK), jnp.int32),
            pltpu.VMEM((k_tile, CHUNK), jnp.int32),
            pltpu.VMEM((2 * NBUF, CHUNK, 32), jnp.float32),
            pltpu.VMEM_SHARED((N_PAD, 32), jnp.float32),
            pltpu.VMEM_SHARED((N_PAD, 32), jnp.float32),
        ] + [pltpu.SemaphoreType.DMA] * (4 * NBUF),
        compiler_params=pltpu.CompilerParams(use_tc_tiling_on_sc=False),
    )
    def prop1_k(ua_hbm, ub_hbm, ei_hbm, out_hbm,
                src_v, dst_v, rows_v, acc_sh, u_sh, *sems):
        gsem = sems[:2 * NBUF]
        ssem = sems[2 * NBUF:]
        c = lax.axis_index("c")
        s = lax.axis_index("s")
        r0 = pl.multiple_of(s * ROWS_PER_TILE, 8)
        pltpu.sync_copy(ei_hbm.at[0, pl.ds(s * k_tile, k_tile)], src_v)
        pltpu.sync_copy(ei_hbm.at[1, pl.ds(s * k_tile, k_tile)], dst_v)

        @pl.when(c == 0)
        def _():
            _stage_rows(ua_hbm, u_sh, s)
            _stage_rows(ua_hbm, acc_sh, s)

        @pl.when(c == 1)
        def _():
            _stage_rows(ub_hbm, u_sh, s)
            _stage_rows(ub_hbm, acc_sh, s)

        plsc.subcore_barrier()
        _gather_scatter_loop(u_sh, acc_sh, src_v, dst_v, rows_v, gsem, ssem,
                             k_tile)
        plsc.subcore_barrier()
        pltpu.sync_copy(acc_sh.at[pl.ds(r0, ROWS_PER_TILE)],
                        out_hbm.at[c, pl.ds(r0, ROWS_PER_TILE)])

    return prop1_k(ua, ub, ei_p)


def _prop2_call(u, ei_p, zeros):
    k_tile = N_CHUNKS // NW

    @functools.partial(
        pl.kernel,
        mesh=_mesh(),
        out_type=jax.ShapeDtypeStruct((NCORE, N_PAD, 32), jnp.float32),
        scratch_types=[
            pltpu.VMEM((k_tile, CHUNK), jnp.int32),
            pltpu.VMEM((k_tile, CHUNK), jnp.int32),
            pltpu.VMEM((2 * NBUF, CHUNK, 32), jnp.float32),
            pltpu.VMEM_SHARED((N_PAD, 32), jnp.float32),
            pltpu.VMEM_SHARED((N_PAD, 32), jnp.float32),
        ] + [pltpu.SemaphoreType.DMA] * (4 * NBUF),
        compiler_params=pltpu.CompilerParams(use_tc_tiling_on_sc=False),
    )
    def prop2_k(u_hbm, ei_hbm, zeros_hbm, out_hbm,
                src_v, dst_v, rows_v, acc_sh, u_sh, *sems):
        gsem = sems[:2 * NBUF]
        ssem = sems[2 * NBUF:]
        c = lax.axis_index("c")
        s = lax.axis_index("s")
        wid = c * NSUB + s
        r0 = pl.multiple_of(s * ROWS_PER_TILE, 8)
        pltpu.sync_copy(ei_hbm.at[0, pl.ds(wid * k_tile, k_tile)], src_v)
        pltpu.sync_copy(ei_hbm.at[1, pl.ds(wid * k_tile, k_tile)], dst_v)
        _stage_rows(u_hbm, u_sh, s)

        @pl.when(c == 0)
        def _():
            _stage_rows(u_hbm, acc_sh, s)

        @pl.when(c == 1)
        def _():
            pltpu.sync_copy(zeros_hbm.at[pl.ds(r0, ROWS_PER_TILE)],
                            acc_sh.at[pl.ds(r0, ROWS_PER_TILE)])

        plsc.subcore_barrier()
        _gather_scatter_loop(u_sh, acc_sh, src_v, dst_v, rows_v, gsem, ssem,
                             k_tile)
        plsc.subcore_barrier()
        pltpu.sync_copy(acc_sh.at[pl.ds(r0, ROWS_PER_TILE)],
                        out_hbm.at[c, pl.ds(r0, ROWS_PER_TILE)])

    return prop2_k(u, ei_p, zeros)


def _tc1m_call(x, W1):
    def body(x_ref, w_ref, h_ref):
        h_ref[...] = jnp.dot(x_ref[...], w_ref[...],
                             preferred_element_type=jnp.float32)

    return pl.pallas_call(
        body,
        grid=(N_NODES // BLK,),
        in_specs=[
            pl.BlockSpec((BLK, 128), lambda i: (i, 0)),
            pl.BlockSpec((128, 64), lambda i: (0, 0)),
        ],
        out_specs=pl.BlockSpec((BLK, 64), lambda i: (i, 0)),
        out_shape=jax.ShapeDtypeStruct((N_NODES, 64), jnp.float32),
    )(x, W1)


def _tc1s_call(bins, h):
    rows = BLK * 16 // 128
    all_rows = N_PAD * 16 // 128

    def body(bins_ref, h_ref, ua_ref, ub_ref, dis_ref):
        i = pl.program_id(0)
        br = (bins_ref[0, pl.ds(i * rows, rows), :]
              + bins_ref[1, pl.ds(i * rows, rows), :])
        deg = br.reshape(rows, 8, 16)[:, :, 0].reshape(BLK, 1) + 1.0
        dis = lax.rsqrt(deg)
        u = h_ref[...] * dis
        ua_ref[...] = u[:, :32]
        ub_ref[...] = u[:, 32:]
        dis_ref[...] = dis

    return pl.pallas_call(
        body,
        grid=(N_NODES // BLK,),
        in_specs=[
            pl.BlockSpec((NCORE, all_rows, 128), lambda i: (0, 0, 0)),
            pl.BlockSpec((BLK, 64), lambda i: (i, 0)),
        ],
        out_specs=[
            pl.BlockSpec((BLK, 32), lambda i: (i, 0)),
            pl.BlockSpec((BLK, 32), lambda i: (i, 0)),
            pl.BlockSpec((BLK, 1), lambda i: (i, 0)),
        ],
        out_shape=[
            jax.ShapeDtypeStruct((N_NODES, 32), jnp.float32),
            jax.ShapeDtypeStruct((N_NODES, 32), jnp.float32),
            jax.ShapeDtypeStruct((N_NODES, 1), jnp.float32),
        ],
    )(bins, h)


def _tc2_call(p1, dis, b1, W2):
    def body(p_ref, dis_ref, b1_ref, w2_ref, u2_ref):
        dis = dis_ref[...]
        oa = jnp.maximum(p_ref[0] * dis + b1_ref[:, :32], 0.0)
        ob = jnp.maximum(p_ref[1] * dis + b1_ref[:, 32:], 0.0)
        u2 = (jnp.dot(oa, w2_ref[0], preferred_element_type=jnp.float32)
              + jnp.dot(ob, w2_ref[1], preferred_element_type=jnp.float32))
        u2_ref[...] = u2 * dis

    return pl.pallas_call(
        body,
        grid=(N_NODES // BLK,),
        in_specs=[
            pl.BlockSpec((NCORE, BLK, 32), lambda i: (0, i, 0)),
            pl.BlockSpec((BLK, 1), lambda i: (i, 0)),
            pl.BlockSpec((1, 64), lambda i: (0, 0)),
            pl.BlockSpec((2, 32, 32), lambda i: (0, 0, 0)),
        ],
        out_specs=pl.BlockSpec((BLK, 32), lambda i: (i, 0)),
        out_shape=jax.ShapeDtypeStruct((N_NODES, 32), jnp.float32),
    )(p1, dis, b1, W2.reshape(2, 32, 32))


def _tc3_call(p, dis, b2, batch_r):
    def body(p_ref, dis_ref, b2_ref, bt_ref, out_ref):
        h2 = (p_ref[0] + p_ref[1]) * dis_ref[...] + b2_ref[...]
        bt = bt_ref[0]
        oh = (lax.broadcasted_iota(jnp.int32, (NUM_GRAPHS, BLK), 0)
              == bt).astype(jnp.float32)
        acc = jnp.dot(oh, h2, preferred_element_type=jnp.float32)
        i = pl.program_id(0)

        @pl.when(i == 0)
        def _():
            out_ref[...] = acc

        @pl.when(i != 0)
        def _():
            out_ref[...] += acc

    return pl.pallas_call(
        body,
        grid=(N_NODES // BLK,),
        in_specs=[
            pl.BlockSpec((NCORE, BLK, 32), lambda i: (0, i, 0)),
            pl.BlockSpec((BLK, 1), lambda i: (i, 0)),
            pl.BlockSpec((1, 32), lambda i: (0, 0)),
            pl.BlockSpec((1, 1, BLK), lambda i: (i, 0, 0)),
        ],
        out_specs=pl.BlockSpec((NUM_GRAPHS, 32), lambda i: (0, 0)),
        out_shape=jax.ShapeDtypeStruct((NUM_GRAPHS, 32), jnp.float32),
    )(p, dis, b2, batch_r)


def kernel(x, edge_index, batch, W1, b1, W2, b2):
    pad = CAP - NUM_EDGES
    pad_src = jnp.arange(pad, dtype=jnp.int32) % N_NODES
    pad_dst = N_NODES + (jnp.arange(pad, dtype=jnp.int32) % (N_PAD - N_NODES))
    ei_p = jnp.concatenate(
        [edge_index.astype(jnp.int32), jnp.stack([pad_src, pad_dst])],
        axis=1).reshape(2, N_CHUNKS, CHUNK)
    ones_rows = jnp.ones((CHUNK, 16), jnp.float32)
    z16 = jnp.zeros((N_PAD, 16), jnp.float32)
    z32 = jnp.zeros((N_PAD, 32), jnp.float32)

    h = _tc1m_call(x, W1)
    bins = _deg_call(ei_p, ones_rows, z16)
    u1a, u1b, dis = _tc1s_call(bins.reshape(NCORE, N_PAD * 16 // 128, 128), h)
    p1 = _prop1_call(u1a, u1b, ei_p)
    u2 = _tc2_call(p1, dis, b1.reshape(1, 64), W2)
    p2 = _prop2_call(u2, ei_p, z32)
    out = _tc3_call(p2, dis, b2.reshape(1, 32),
                    batch.astype(jnp.int32).reshape(N_NODES // BLK, 1, BLK))
    return out


# speedup vs baseline: 2.6409x; 1.0005x over previous
import functools

import jax
import jax.numpy as jnp
from jax import lax
from jax.experimental import pallas as pl
from jax.experimental.pallas import tpu as pltpu
from jax.experimental.pallas import tpu_sc as plsc

N_NODES = 10000
NUM_EDGES = 320000
NUM_GRAPHS = 64
NCORE = 2
NSUB = 16
NW = NCORE * NSUB
CHUNK = 128
N_CHUNKS = 2560
NBUF = 4
CAP = N_CHUNKS * CHUNK
N_PAD = 10112
ROWS_PER_TILE = N_PAD // NSUB
BLK = 2000


def _mesh():
    return plsc.VectorSubcoreMesh(core_axis_name="c", subcore_axis_name="s")


def _stage_rows(u_hbm, u_sh, s):
    r0 = pl.multiple_of(s * ROWS_PER_TILE, 8)
    nfull = N_NODES // ROWS_PER_TILE
    rem = N_NODES - nfull * ROWS_PER_TILE

    @pl.when(s < nfull)
    def _():
        pltpu.sync_copy(u_hbm.at[pl.ds(r0, ROWS_PER_TILE)],
                        u_sh.at[pl.ds(r0, ROWS_PER_TILE)])

    @pl.when(s == nfull)
    def _():
        rr = pl.multiple_of(nfull * ROWS_PER_TILE, 8)
        pltpu.sync_copy(u_hbm.at[pl.ds(rr, rem)], u_sh.at[pl.ds(rr, rem)])


def _gather_scatter_loop(u_src, acc_sh, src_v, dst_v, rows_v, gsem, ssem,
                         k_tile):
    nstep = k_tile // (2 * NBUF)

    def fire_gather(slot, j):
        pltpu.async_copy(u_src.at[src_v.at[j]], rows_v.at[slot], gsem[slot])

    def wait_gather(slot, j):
        pltpu.make_async_copy(u_src.at[src_v.at[j]], rows_v.at[slot],
                              gsem[slot]).wait()

    for b in range(2 * NBUF):
        fire_gather(b, b)

    def body(t, carry):
        j0 = 2 * NBUF * t
        for half in range(2):
            off = half * NBUF
            for b in range(NBUF):
                wait_gather(off + b, j0 + off + b)
            scps = [
                pltpu.async_copy(rows_v.at[off + b],
                                 acc_sh.at[dst_v.at[j0 + off + b]],
                                 ssem[off + b], add=True)
                for b in range(NBUF)
            ]
            for d_ in scps:
                d_.wait()

            @pl.when(t < nstep - 1)
            def _():
                for b in range(NBUF):
                    fire_gather(off + b, j0 + 2 * NBUF + off + b)

        return carry

    lax.fori_loop(0, nstep, body, 0)


def _deg_call(ei_p, ones_rows, zeros16):
    k_tile = N_CHUNKS // NW

    @functools.partial(
        pl.kernel,
        mesh=_mesh(),
        out_type=jax.ShapeDtypeStruct((NCORE, N_PAD, 16), jnp.float32),
        scratch_types=[
            pltpu.VMEM((k_tile, CHUNK), jnp.int32),
            pltpu.VMEM((CHUNK, 16), jnp.float32),
            pltpu.VMEM_SHARED((N_PAD, 16), jnp.float32),
            pltpu.SemaphoreType.DMA,
        ],
        compiler_params=pltpu.CompilerParams(use_tc_tiling_on_sc=False),
    )
    def deg_k(ei_hbm, ones_hbm, zeros_hbm, out_hbm, idx_v, ones_v, bins_sh,
              sem):
        c = lax.axis_index("c")
        s = lax.axis_index("s")
        wid = c * NSUB + s
        r0 = pl.multiple_of(s * ROWS_PER_TILE, 8)
        pltpu.sync_copy(zeros_hbm.at[pl.ds(r0, ROWS_PER_TILE)],
                        bins_sh.at[pl.ds(r0, ROWS_PER_TILE)])
        pltpu.sync_copy(ones_hbm, ones_v)
        pltpu.sync_copy(ei_hbm.at[1, pl.ds(wid * k_tile, k_tile)], idx_v)
        plsc.subcore_barrier()

        def body(t, carry):
            for b in range(8):
                pltpu.async_copy(ones_v, bins_sh.at[idx_v.at[t * 8 + b]],
                                 sem, add=True)

            @pl.when(t > 0)
            def _():
                for b in range(8):
                    pltpu.make_async_copy(
                        ones_v, bins_sh.at[idx_v.at[b]], sem).wait()

            return carry

        lax.fori_loop(0, k_tile // 8, body, 0)
        for b in range(8):
            pltpu.make_async_copy(ones_v, bins_sh.at[idx_v.at[b]], sem).wait()
        plsc.subcore_barrier()
        pltpu.sync_copy(bins_sh.at[pl.ds(r0, ROWS_PER_TILE)],
                        out_hbm.at[c, pl.ds(r0, ROWS_PER_TILE)])

    return deg_k(ei_p, ones_rows, zeros16)


def _prop1_call(ua, ub, ei_p):
    k_tile = N_CHUNKS // NSUB

    @functools.partial(
        pl.kernel,
        mesh=_mesh(),
        out_type=jax.ShapeDtypeStruct((NCORE, N_PAD, 32), jnp.float32),
        scratch_types=[
            pltpu.VMEM((k_tile, CHUNK), jnp.int32),
            pltpu.VMEM((k_tile, CHUNK), jnp.int32),
            pltpu.VMEM((2 * NBUF, CHUNK, 32), jnp.float32),
            pltpu.VMEM_SHARED((N_PAD, 32), jnp.float32),
            pltpu.VMEM_SHARED((N_PAD, 32), jnp.float32),
        ] + [pltpu.SemaphoreType.DMA] * (4 * NBUF),
        compiler_params=pltpu.CompilerParams(use_tc_tiling_on_sc=False),
    )
    def prop1_k(ua_hbm, ub_hbm, ei_hbm, out_hbm,
                src_v, dst_v, rows_v, acc_sh, u_sh, *sems):
        gsem = sems[:2 * NBUF]
        ssem = sems[2 * NBUF:]
        c = lax.axis_index("c")
        s = lax.axis_index("s")
        r0 = pl.multiple_of(s * ROWS_PER_TILE, 8)
        pltpu.sync_copy(ei_hbm.at[0, pl.ds(s * k_tile, k_tile)], src_v)
        pltpu.sync_copy(ei_hbm.at[1, pl.ds(s * k_tile, k_tile)], dst_v)

        @pl.when(c == 0)
        def _():
            _stage_rows(ua_hbm, u_sh, s)
            _stage_rows(ua_hbm, acc_sh, s)

        @pl.when(c == 1)
        def _():
            _stage_rows(ub_hbm, u_sh, s)
            _stage_rows(ub_hbm, acc_sh, s)

        plsc.subcore_barrier()
        _gather_scatter_loop(u_sh, acc_sh, src_v, dst_v, rows_v, gsem, ssem,
                             k_tile)
        plsc.subcore_barrier()
        pltpu.sync_copy(acc_sh.at[pl.ds(r0, ROWS_PER_TILE)],
                        out_hbm.at[c, pl.ds(r0, ROWS_PER_TILE)])

    return prop1_k(ua, ub, ei_p)


def _prop2_call(u, ei_p, zeros):
    k_tile = N_CHUNKS // NW

    @functools.partial(
        pl.kernel,
        mesh=_mesh(),
        out_type=jax.ShapeDtypeStruct((NCORE, N_PAD, 32), jnp.float32),
        scratch_types=[
            pltpu.VMEM((k_tile, CHUNK), jnp.int32),
            pltpu.VMEM((k_tile, CHUNK), jnp.int32),
            pltpu.VMEM((2 * NBUF, CHUNK, 32), jnp.float32),
            pltpu.VMEM_SHARED((N_PAD, 32), jnp.float32),
            pltpu.VMEM_SHARED((N_PAD, 32), jnp.float32),
        ] + [pltpu.SemaphoreType.DMA] * (4 * NBUF),
        compiler_params=pltpu.CompilerParams(use_tc_tiling_on_sc=False),
    )
    def prop2_k(u_hbm, ei_hbm, zeros_hbm, out_hbm,
                src_v, dst_v, rows_v, acc_sh, u_sh, *sems):
        gsem = sems[:2 * NBUF]
        ssem = sems[2 * NBUF:]
        c = lax.axis_index("c")
        s = lax.axis_index("s")
        wid = c * NSUB + s
        r0 = pl.multiple_of(s * ROWS_PER_TILE, 8)
        pltpu.sync_copy(ei_hbm.at[0, pl.ds(wid * k_tile, k_tile)], src_v)
        pltpu.sync_copy(ei_hbm.at[1, pl.ds(wid * k_tile, k_tile)], dst_v)
        _stage_rows(u_hbm, u_sh, s)

        @pl.when(c == 0)
        def _():
            _stage_rows(u_hbm, acc_sh, s)

        @pl.when(c == 1)
        def _():
            pltpu.sync_copy(zeros_hbm.at[pl.ds(r0, ROWS_PER_TILE)],
                            acc_sh.at[pl.ds(r0, ROWS_PER_TILE)])

        plsc.subcore_barrier()
        _gather_scatter_loop(u_sh, acc_sh, src_v, dst_v, rows_v, gsem, ssem,
                             k_tile)
        plsc.subcore_barrier()
        pltpu.sync_copy(acc_sh.at[pl.ds(r0, ROWS_PER_TILE)],
                        out_hbm.at[c, pl.ds(r0, ROWS_PER_TILE)])

    return prop2_k(u, ei_p, zeros)


def _tc1m_call(x, W1):
    def body(x_ref, w_ref, h_ref):
        h_ref[...] = jnp.dot(x_ref[...], w_ref[...],
                             preferred_element_type=jnp.float32)

    return pl.pallas_call(
        body,
        grid=(N_NODES // BLK,),
        in_specs=[
            pl.BlockSpec((BLK, 128), lambda i: (i, 0)),
            pl.BlockSpec((128, 64), lambda i: (0, 0)),
        ],
        out_specs=pl.BlockSpec((BLK, 64), lambda i: (i, 0)),
        out_shape=jax.ShapeDtypeStruct((N_NODES, 64), jnp.float32),
    )(x, W1)


def _tc1s_call(bins, h):
    rows = BLK * 16 // 128
    all_rows = N_PAD * 16 // 128

    def body(bins_ref, h_ref, ua_ref, ub_ref, dis_ref):
        i = pl.program_id(0)
        br = (bins_ref[0, pl.ds(i * rows, rows), :]
              + bins_ref[1, pl.ds(i * rows, rows), :])
        deg = br.reshape(rows, 8, 16)[:, :, 0].reshape(BLK, 1) + 1.0
        dis = lax.rsqrt(deg)
        u = h_ref[...] * dis
        ua_ref[...] = u[:, :32]
        ub_ref[...] = u[:, 32:]
        dis_ref[...] = dis

    return pl.pallas_call(
        body,
        grid=(N_NODES // BLK,),
        in_specs=[
            pl.BlockSpec((NCORE, all_rows, 128), lambda i: (0, 0, 0)),
            pl.BlockSpec((BLK, 64), lambda i: (i, 0)),
        ],
        out_specs=[
            pl.BlockSpec((BLK, 32), lambda i: (i, 0)),
            pl.BlockSpec((BLK, 32), lambda i: (i, 0)),
            pl.BlockSpec((BLK, 1), lambda i: (i, 0)),
        ],
        out_shape=[
            jax.ShapeDtypeStruct((N_NODES, 32), jnp.float32),
            jax.ShapeDtypeStruct((N_NODES, 32), jnp.float32),
            jax.ShapeDtypeStruct((N_NODES, 1), jnp.float32),
        ],
    )(bins, h)


P_ROWS = BLK * 32 // 128
P_ALL = N_PAD * 32 // 128


def _tc2_call(p1, dis, b1, W2):
    def body(p_ref, dis_ref, b1_ref, w2_ref, u2_ref):
        pa = p_ref[0]
        pb = p_ref[1]
        dis = dis_ref[...]
        oa = jnp.maximum(pa * dis + b1_ref[:, :32], 0.0)
        ob = jnp.maximum(pb * dis + b1_ref[:, 32:], 0.0)
        u2 = (jnp.dot(oa, w2_ref[0], preferred_element_type=jnp.float32)
              + jnp.dot(ob, w2_ref[1], preferred_element_type=jnp.float32))
        u2_ref[...] = u2 * dis

    return pl.pallas_call(
        body,
        grid=(N_NODES // BLK,),
        in_specs=[
            pl.BlockSpec((NCORE, BLK, 32), lambda i: (0, i, 0)),
            pl.BlockSpec((BLK, 1), lambda i: (i, 0)),
            pl.BlockSpec((1, 64), lambda i: (0, 0)),
            pl.BlockSpec((2, 32, 32), lambda i: (0, 0, 0)),
        ],
        out_specs=pl.BlockSpec((BLK, 32), lambda i: (i, 0)),
        out_shape=jax.ShapeDtypeStruct((N_NODES, 32), jnp.float32),
    )(p1, dis, b1, W2.reshape(2, 32, 32))


def _tc3_call(p, dis, b2, batch_r):
    def body(p_ref, dis_ref, b2_ref, bt_ref, out_ref):
        i = pl.program_id(0)
        h2 = (p_ref[0] + p_ref[1]) * dis_ref[...] + b2_ref[...]
        bt = bt_ref[0]
        oh = (lax.broadcasted_iota(jnp.int32, (NUM_GRAPHS, BLK), 0)
              == bt).astype(jnp.float32)
        acc = jnp.dot(oh, h2, preferred_element_type=jnp.float32)

        @pl.when(i == 0)
        def _():
            out_ref[...] = acc

        @pl.when(i != 0)
        def _():
            out_ref[...] += acc

    return pl.pallas_call(
        body,
        grid=(N_NODES // BLK,),
        in_specs=[
            pl.BlockSpec((NCORE, BLK, 32), lambda i: (0, i, 0)),
            pl.BlockSpec((BLK, 1), lambda i: (i, 0)),
            pl.BlockSpec((1, 32), lambda i: (0, 0)),
            pl.BlockSpec((1, 1, BLK), lambda i: (i, 0, 0)),
        ],
        out_specs=pl.BlockSpec((NUM_GRAPHS, 32), lambda i: (0, 0)),
        out_shape=jax.ShapeDtypeStruct((NUM_GRAPHS, 32), jnp.float32),
    )(p, dis, b2, batch_r)


def kernel(x, edge_index, batch, W1, b1, W2, b2):
    pad = CAP - NUM_EDGES
    pad_src = jnp.arange(pad, dtype=jnp.int32) % N_NODES
    pad_dst = N_NODES + (jnp.arange(pad, dtype=jnp.int32) % (N_PAD - N_NODES))
    ei_p = jnp.concatenate(
        [edge_index.astype(jnp.int32), jnp.stack([pad_src, pad_dst])],
        axis=1).reshape(2, N_CHUNKS, CHUNK)
    ones_rows = jnp.ones((CHUNK, 16), jnp.float32)
    z16 = jnp.zeros((N_PAD, 16), jnp.float32)
    z32 = jnp.zeros((N_PAD, 32), jnp.float32)

    h = _tc1m_call(x, W1)
    bins = _deg_call(ei_p, ones_rows, z16)
    u1a, u1b, dis = _tc1s_call(bins.reshape(NCORE, N_PAD * 16 // 128, 128), h)
    p1 = _prop1_call(u1a, u1b, ei_p)
    u2 = _tc2_call(p1, dis, b1.reshape(1, 64), W2)
    p2 = _prop2_call(u2, ei_p, z32)
    out = _tc3_call(p2, dis, b2.reshape(1, 32),
                    batch.astype(jnp.int32).reshape(N_NODES // BLK, 1, BLK))
    return out
